# Initial kernel scaffold; baseline (speedup 1.0000x reference)
import jax
import jax.numpy as jnp
from jax.experimental import pallas as pl


def kernel(x, edge_index, edge_attr, l1_lw, l1_lb, l1_rw, l1_rb, l1_ew, l1_att, l1_bias, l2_lw, l2_lb, l2_rw, l2_rb, l2_ew, l2_att, l2_bias, c1_w, c1_b, c2_w, c2_b):
    raise NotImplementedError("write your pallas kernel here")


# speedup vs baseline: 10.3538x; 10.3538x over previous
import functools

import jax
import jax.numpy as jnp
import numpy as np
from jax import lax
from jax.experimental import pallas as pl
from jax.experimental.pallas import tpu as pltpu
from jax.experimental.pallas import tpu_sc as plsc

N = 10000
E = 320000
ND = 128
ED = 16
HID = 64
NC = 2
NS = 16
CB = 80
BN = 1000
BEP = 2000

F32 = jnp.float32
I32 = jnp.int32

_MESH = dict(core_axis_name="c", subcore_axis_name="s")

_EPW = E // (NC * NS)
_EPT = E // NS
_SEG = 624
_SEGC = 104
_TAIL = N - NS * _SEG

_DNUMS = lax.GatherDimensionNumbers(
    offset_dims=(), collapsed_slice_dims=(0,), start_index_map=(0,))


def _allsum(v):
    lanes = lax.iota(I32, 16)
    for k in range(4):
        p = lax.bitwise_xor(lanes, 1 << k)
        v = v + lax.gather(v, p[:, None], _DNUMS, slice_sizes=(1,),
                           mode=lax.GatherScatterMode.PROMISE_IN_BOUNDS)
    return v


def _zeros16():
    return jnp.zeros((16,), F32)


def _m8(v):
    return pl.multiple_of(v, 8)


def _zero_vmem(buf, w):
    z16 = _zeros16()

    def zr(i, _):
        for q in range(w // 16):
            buf[i, pl.ds(q * 16, 16)] = z16
        return 0
    lax.fori_loop(0, buf.shape[0], zr, 0)


def _zero_shared(zb, sh, sid):
    for t in range(_SEG // _SEGC):
        pltpu.sync_copy(zb, sh.at[pl.ds(_m8(sid * _SEG + t * _SEGC), _SEGC)])

    @pl.when(sid == NS - 1)
    def _():
        pltpu.sync_copy(zb.at[pl.ds(0, _TAIL)], sh.at[pl.ds(NS * _SEG, _TAIL)])


def _copy_out_shared(sh, hbm, cid, sid):
    pltpu.sync_copy(sh.at[pl.ds(_m8(sid * _SEG), _SEG)],
                    hbm.at[pl.ds(_m8(cid * N + sid * _SEG), _SEG)])

    @pl.when(sid == NS - 1)
    def _():
        pltpu.sync_copy(sh.at[pl.ds(NS * _SEG, _TAIL)],
                        hbm.at[pl.ds(_m8(cid * N + NS * _SEG), _TAIL)])


def _t1_body(x_ref, lwT_ref, lb_ref, rwT_ref, rb_ref, c_ref):
    xv = x_ref[...]
    xl = jnp.dot(xv, lwT_ref[0], preferred_element_type=F32) + lb_ref[0]
    xr = jnp.dot(xv, rwT_ref[0], preferred_element_type=F32) + rb_ref[0]
    c_ref[...] = jnp.concatenate([xl, xr], axis=1)[None]


def _t1(x, lwT, lb, rwT, rb):
    return pl.pallas_call(
        _t1_body,
        grid=(4, N // BN),
        in_specs=[
            pl.BlockSpec((BN, ND), lambda c, i: (i, 0)),
            pl.BlockSpec((1, ND, HID), lambda c, i: (c, 0, 0)),
            pl.BlockSpec((1, 1, HID), lambda c, i: (c, 0, 0)),
            pl.BlockSpec((1, ND, HID), lambda c, i: (c, 0, 0)),
            pl.BlockSpec((1, 1, HID), lambda c, i: (c, 0, 0)),
        ],
        out_specs=pl.BlockSpec((1, BN, 128), lambda c, i: (c, i, 0)),
        out_shape=jax.ShapeDtypeStruct((4, N, 128), F32),
    )(x, lwT, lb, rwT, rb)


def _t2a_body(eae_ref, eao_ref, ewT_ref, ep_ref):
    pe = jnp.dot(eae_ref[...], ewT_ref[0], preferred_element_type=F32)
    po = jnp.dot(eao_ref[...], ewT_ref[0], preferred_element_type=F32)
    ep_ref[...] = jnp.concatenate([pe, po], axis=1)[None]


def _t2a(eae, eao, ewT):
    return pl.pallas_call(
        _t2a_body,
        grid=(4, (E // 2) // BEP),
        in_specs=[
            pl.BlockSpec((BEP, ED), lambda c, i: (i, 0)),
            pl.BlockSpec((BEP, ED), lambda c, i: (i, 0)),
            pl.BlockSpec((1, ED, HID), lambda c, i: (c, 0, 0)),
        ],
        out_specs=pl.BlockSpec((1, BEP, 128), lambda c, i: (c, i, 0)),
        out_shape=jax.ShapeDtypeStruct((4, E // 2, 128), F32),
    )(eae, eao, ewT)


def _t2b_body(eae_ref, eao_ref, w2T_ref, weT_ref, cb_ref, ep2_ref, eec_ref):
    eav = eae_ref[...]
    eov = eao_ref[...]
    ep2_ref[...] = jnp.concatenate(
        [jnp.dot(eav, w2T_ref[...], preferred_element_type=F32),
         jnp.dot(eov, w2T_ref[...], preferred_element_type=F32)], axis=1)
    eec_ref[...] = jnp.concatenate(
        [jnp.dot(eav, weT_ref[...], preferred_element_type=F32) + cb_ref[...],
         jnp.dot(eov, weT_ref[...], preferred_element_type=F32) + cb_ref[...]],
        axis=1)


def _t2b(eae, eao, w2T, weT, cb):
    return pl.pallas_call(
        _t2b_body,
        grid=((E // 2) // BEP,),
        in_specs=[
            pl.BlockSpec((BEP, ED), lambda i: (i, 0)),
            pl.BlockSpec((BEP, ED), lambda i: (i, 0)),
            pl.BlockSpec((ED, HID), lambda i: (0, 0)),
            pl.BlockSpec((ED, HID), lambda i: (0, 0)),
            pl.BlockSpec((1, HID), lambda i: (0, 0)),
        ],
        out_specs=[pl.BlockSpec((BEP, 128), lambda i: (i, 0))] * 2,
        out_shape=[jax.ShapeDtypeStruct((E // 2, 128), F32)] * 2,
    )(eae, eao, w2T, weT, cb)


def _s0_body(dst_hbm, ea_hbm, acc_hbm,
             dstv, eab, wbuf, zb, acc_sh, sem):
    cid = lax.axis_index("c")
    sid = lax.axis_index("s")
    wid = cid * NS + sid
    lanes = lax.iota(I32, 16)
    mask0 = jnp.where(lanes == 0, 1.0, 0.0).astype(F32)

    _zero_vmem(wbuf, 128)

    def fill(i, _):
        wbuf[i, pl.ds(16, 16)] = mask0
        return 0
    lax.fori_loop(0, CB, fill, 0)
    _zero_vmem(zb, 128)
    _zero_shared(zb, acc_sh, sid)
    plsc.subcore_barrier()

    def chunk(k, _):
        base = _m8(wid * _EPW + k * CB)
        pltpu.sync_copy(dst_hbm.at[pl.ds(base, CB)], dstv)
        pltpu.sync_copy(ea_hbm.at[pl.ds(_m8(base * ED), CB * ED)], eab)

        def edge(j, _):
            wbuf[j, pl.ds(0, 16)] = eab[pl.ds(j * ED, 16)]
            return 0
        lax.fori_loop(0, CB, edge, 0)
        pltpu.sync_copy(wbuf, acc_sh.at[dstv], add=True)
        return 0
    lax.fori_loop(0, _EPW // CB, chunk, 0)
    plsc.subcore_barrier()

    _copy_out_shared(acc_sh, acc_hbm, cid, sid)


def _s0(dst, eaf):
    return pl.kernel(
        _s0_body,
        out_type=jax.ShapeDtypeStruct((NC * N, 128), F32),
        mesh=plsc.VectorSubcoreMesh(**_MESH),
        scratch_types=[
            pltpu.VMEM((CB,), I32),
            pltpu.VMEM((CB * ED,), F32),
            pltpu.VMEM((CB, 128), F32),
            pltpu.VMEM((_SEGC, 128), F32),
            pltpu.VMEM_SHARED((N, 128), F32),
            pltpu.SemaphoreType.DMA,
        ],
    )(dst, eaf)


def _s1h_body(t, src_hbm, dst_hbm, tbl_hbm, ep_hbm, att_hbm,
              acc_hbm,
              idxl, idxr, dstv, srow, drow, epb, wbuf, attb, zb,
              acc_sh, sem):
    cid = lax.axis_index("c")
    sid = lax.axis_index("s")
    lanes = lax.iota(I32, 16)
    mask0 = jnp.where(lanes == 0, 1.0, 0.0).astype(F32)
    head = 2 * t + cid

    pltpu.sync_copy(att_hbm, attb)
    att_vecs = [attb[pl.ds(head * HID + q * 16, 16)] for q in range(4)]

    _zero_vmem(wbuf, 128)
    _zero_vmem(zb, 128)
    _zero_shared(zb, acc_sh, sid)
    plsc.subcore_barrier()

    ioff = head * N

    def chunk(k, _):
        base = _m8(sid * _EPT + k * CB)
        pltpu.sync_copy(src_hbm.at[pl.ds(base, CB)], idxl)
        pltpu.sync_copy(dst_hbm.at[pl.ds(base, CB)], dstv)
        for g in range(CB // 16):
            sl = pl.ds(g * 16, 16)
            idxl[sl] = idxl[sl] + ioff
            idxr[sl] = dstv[sl] + ioff
        cpl = pltpu.async_copy(tbl_hbm.at[idxl], srow, sem)
        cpr = pltpu.async_copy(tbl_hbm.at[idxr], drow, sem)
        cpe = pltpu.async_copy(
            ep_hbm.at[pl.ds(_m8((head * E + base) // 2), CB // 2)], epb, sem)
        cpl.wait()
        cpr.wait()
        cpe.wait()

        def pair(p, _):
            for u in range(2):
                j = 2 * p + u
                acc = _zeros16()
                for q in range(4):
                    s = (srow[j, pl.ds(q * 16, 16)]
                         + drow[j, pl.ds(64 + q * 16, 16)]
                         + epb[p, pl.ds(u * 64 + q * 16, 16)])
                    m = jnp.maximum(s, 0.2 * s)
                    acc = acc + m * att_vecs[q]
                ex = jnp.exp(_allsum(acc))
                for q in range(4):
                    sl = pl.ds(q * 16, 16)
                    wbuf[j, sl] = srow[j, sl] * ex
                wbuf[j, pl.ds(64, 16)] = ex * mask0
            return 0
        lax.fori_loop(0, CB // 2, pair, 0)
        pltpu.sync_copy(wbuf, acc_sh.at[dstv], add=True)
        return 0
    lax.fori_loop(0, _EPT // CB, chunk, 0)
    plsc.subcore_barrier()

    _copy_out_shared(acc_sh, acc_hbm, cid, sid)


def _s1h(t, src, dst, tbl, ep, attf):
    return pl.kernel(
        functools.partial(_s1h_body, t),
        out_type=jax.ShapeDtypeStruct((NC * N, 128), F32),
        mesh=plsc.VectorSubcoreMesh(**_MESH),
        scratch_types=[
            pltpu.VMEM((CB,), I32),
            pltpu.VMEM((CB,), I32),
            pltpu.VMEM((CB,), I32),
            pltpu.VMEM((CB, 128), F32),
            pltpu.VMEM((CB, 128), F32),
            pltpu.VMEM((CB // 2, 128), F32),
            pltpu.VMEM((CB, 128), F32),
            pltpu.VMEM((256,), F32),
            pltpu.VMEM((_SEGC, 128), F32),
            pltpu.VMEM_SHARED((N, 128), F32),
            pltpu.SemaphoreType.DMA,
        ],
    )(src, dst, tbl, ep, attf)


def _s2_body(src_hbm, dst_hbm, tbl_hbm, ep_hbm, att_hbm,
             acc_hbm,
             idxl, idxr, srow, drow, epb, wbuf, attb, zb,
             acc_sh, sem):
    cid = lax.axis_index("c")
    sid = lax.axis_index("s")
    wid = cid * NS + sid
    lanes = lax.iota(I32, 16)
    mask0 = jnp.where(lanes == 0, 1.0, 0.0).astype(F32)

    pltpu.sync_copy(att_hbm, attb)
    att_vecs = [attb[pl.ds(q * 16, 16)] for q in range(4)]

    _zero_vmem(wbuf, 128)
    _zero_vmem(zb, 128)
    _zero_shared(zb, acc_sh, sid)
    plsc.subcore_barrier()

    def chunk(k, _):
        base = _m8(wid * _EPW + k * CB)
        pltpu.sync_copy(src_hbm.at[pl.ds(base, CB)], idxl)
        pltpu.sync_copy(dst_hbm.at[pl.ds(base, CB)], idxr)
        cpl = pltpu.async_copy(tbl_hbm.at[idxl], srow, sem)
        cpr = pltpu.async_copy(tbl_hbm.at[idxr], drow, sem)
        cpe = pltpu.async_copy(ep_hbm.at[pl.ds(_m8(base // 2), CB // 2)], epb, sem)
        cpl.wait()
        cpr.wait()
        cpe.wait()

        def pair(p, _):
            for u in range(2):
                j = 2 * p + u
                acc = _zeros16()
                for q in range(4):
                    s = (srow[j, pl.ds(q * 16, 16)]
                         + drow[j, pl.ds(64 + q * 16, 16)]
                         + epb[p, pl.ds(u * 64 + q * 16, 16)])
                    m = jnp.maximum(s, 0.2 * s)
                    acc = acc + m * att_vecs[q]
                ex = jnp.exp(_allsum(acc))
                for q in range(4):
                    sl = pl.ds(q * 16, 16)
                    wbuf[j, sl] = srow[j, sl] * ex
                wbuf[j, pl.ds(64, 16)] = ex * mask0
            return 0
        lax.fori_loop(0, CB // 2, pair, 0)
        pltpu.sync_copy(wbuf, acc_sh.at[idxr], add=True)
        return 0
    lax.fori_loop(0, _EPW // CB, chunk, 0)
    plsc.subcore_barrier()

    _copy_out_shared(acc_sh, acc_hbm, cid, sid)


def _s2(src, dst, tbl, ep, attf):
    return pl.kernel(
        _s2_body,
        out_type=jax.ShapeDtypeStruct((NC * N, 128), F32),
        mesh=plsc.VectorSubcoreMesh(**_MESH),
        scratch_types=[
            pltpu.VMEM((CB,), I32),
            pltpu.VMEM((CB,), I32),
            pltpu.VMEM((CB, 128), F32),
            pltpu.VMEM((CB, 128), F32),
            pltpu.VMEM((CB // 2, 128), F32),
            pltpu.VMEM((CB, 128), F32),
            pltpu.VMEM((64,), F32),
            pltpu.VMEM((_SEGC, 128), F32),
            pltpu.VMEM_SHARED((N, 128), F32),
            pltpu.SemaphoreType.DMA,
        ],
    )(src, dst, tbl, ep, attf)


def _s3_body(src_hbm, dst_hbm, g_hbm, eec_hbm, cw_hbm,
             lg_hbm,
             idxl, idxr, srow, drow, ecb, wb, lbuf, sem):
    cid = lax.axis_index("c")
    sid = lax.axis_index("s")
    wid = cid * NS + sid
    lanes = lax.iota(I32, 16)

    pltpu.sync_copy(cw_hbm, wb)
    w_vecs = [wb[pl.ds(q * 16, 16)] for q in range(4)]
    cb_vec = wb[pl.ds(64, 16)]

    def chunk(k, _):
        base = _m8(wid * _EPW + k * CB)
        pltpu.sync_copy(src_hbm.at[pl.ds(base, CB)], idxl)
        pltpu.sync_copy(dst_hbm.at[pl.ds(base, CB)], idxr)
        cpl = pltpu.async_copy(g_hbm.at[idxl], srow, sem)
        cpr = pltpu.async_copy(g_hbm.at[idxr], drow, sem)
        cpe = pltpu.async_copy(eec_hbm.at[pl.ds(_m8(base // 2), CB // 2)], ecb, sem)
        cpl.wait()
        cpr.wait()
        cpe.wait()

        def group(g, _):
            lvec = cb_vec
            for jj in range(16):
                j = g * 16 + jj
                acc = _zeros16()
                for q in range(4):
                    z = (srow[j, pl.ds(q * 16, 16)]
                         + drow[j, pl.ds(64 + q * 16, 16)]
                         + ecb[(g * 16 + jj) // 2,
                               pl.ds((jj % 2) * 64 + q * 16, 16)])
                    z = jnp.maximum(z, 0.0)
                    acc = acc + z * w_vecs[q]
                a = _allsum(acc)
                mj = jnp.where(lanes == jj, 1.0, 0.0).astype(F32)
                lvec = lvec + a * mj
            lbuf[pl.ds(g * 16, 16)] = lvec
            return 0
        lax.fori_loop(0, CB // 16, group, 0)
        pltpu.sync_copy(lbuf, lg_hbm.at[pl.ds(base, CB)])
        return 0
    lax.fori_loop(0, _EPW // CB, chunk, 0)


def _s3(src, dst, g, eec, cwm):
    return pl.kernel(
        _s3_body,
        out_type=jax.ShapeDtypeStruct((E,), F32),
        mesh=plsc.VectorSubcoreMesh(**_MESH),
        scratch_types=[
            pltpu.VMEM((CB,), I32),
            pltpu.VMEM((CB,), I32),
            pltpu.VMEM((CB, 128), F32),
            pltpu.VMEM((CB, 128), F32),
            pltpu.VMEM((CB // 2, 128), F32),
            pltpu.VMEM((80,), F32),
            pltpu.VMEM((CB,), F32),
            pltpu.SemaphoreType.DMA,
        ],
    )(src, dst, g, eec, cwm)


def _ep1_body(acca_ref, accb_ref, s0_ref, xlr_ref,
              ewT_ref, s16_ref, s64_ref, attm_ref, bias_ref,
              l2lwT_ref, l2lb_ref, l2rwT_ref, l2rb_ref,
              c2_ref, la_ref):
    s0s = s0_ref[0] + s0_ref[1]
    ea = s0s[:, :ED]
    cntv = jnp.dot(s0s, s16_ref[...], preferred_element_type=F32)
    la = ea / jnp.maximum(cntv, 1.0)
    la_ref[...] = la
    lp = jnp.dot(la, ewT_ref[...], preferred_element_type=F32)

    accs = [acca_ref[0], acca_ref[1], accb_ref[0], accb_ref[1]]
    xls = []
    mls = []
    for h in range(4):
        xlh = xlr_ref[h][:, :HID]
        xrh = xlr_ref[h][:, HID:]
        mm = xlh + xrh + lp[:, h * HID:(h + 1) * HID]
        xls.append(xlh)
        mls.append(jnp.maximum(mm, 0.2 * mm))
    al = sum(jnp.dot(mls[h], attm_ref[h * HID:(h + 1) * HID, :],
                     preferred_element_type=F32) for h in range(4))
    exl = jnp.exp(al)
    xl2v = l2lb_ref[...]
    xr2v = l2rb_ref[...]
    for h in range(4):
        exh = exl[:, h:h + 1]
        denh = jnp.dot(accs[h], s64_ref[...], preferred_element_type=F32)
        num = accs[h][:, :HID] + exh * xls[h]
        hv = num / (denh + exh + 1e-16) + bias_ref[:, h * HID:(h + 1) * HID]
        hv = jnp.where(hv > 0, hv, jnp.exp(jnp.minimum(hv, 0.0)) - 1.0)
        xl2v = xl2v + jnp.dot(hv, l2lwT_ref[h * HID:(h + 1) * HID, :],
                              preferred_element_type=F32)
        xr2v = xr2v + jnp.dot(hv, l2rwT_ref[h * HID:(h + 1) * HID, :],
                              preferred_element_type=F32)
    c2_ref[...] = jnp.concatenate([xl2v, xr2v], axis=1)


def _ep1(acca, accb, s0acc, xlr, ewT, s16, s64, attm, bias,
         l2lwT, l2lb, l2rwT, l2rb):
    def full(shape):
        return pl.BlockSpec(shape, lambda i, _n=len(shape): (0,) * _n)
    return pl.pallas_call(
        _ep1_body,
        grid=(N // BN,),
        in_specs=[
            pl.BlockSpec((NC, BN, 128), lambda i: (0, i, 0)),
            pl.BlockSpec((NC, BN, 128), lambda i: (0, i, 0)),
            pl.BlockSpec((NC, BN, 128), lambda i: (0, i, 0)),
            pl.BlockSpec((4, BN, 128), lambda i: (0, i, 0)),
            full((ED, 256)),
            full((128, 1)),
            full((128, 1)),
            full((256, 4)),
            full((1, 256)),
            full((256, HID)),
            full((1, HID)),
            full((256, HID)),
            full((1, HID)),
        ],
        out_specs=[pl.BlockSpec((BN, 128), lambda i: (i, 0)),
                   pl.BlockSpec((BN, ED), lambda i: (i, 0))],
        out_shape=[jax.ShapeDtypeStruct((N, 128), F32),
                   jax.ShapeDtypeStruct((N, ED), F32)],
    )(acca, accb, s0acc, xlr, ewT, s16, s64, attm, bias,
      l2lwT, l2lb, l2rwT, l2rb)


def _ep2_body(acc_ref, c2_ref, la_ref,
              ew2T_ref, att2_ref, s64_ref, bias_ref, wsT_ref, wdT_ref,
              g_ref):
    accs = acc_ref[0] + acc_ref[1]
    den = jnp.dot(accs, s64_ref[...], preferred_element_type=F32)
    lp2 = jnp.dot(la_ref[...], ew2T_ref[...], preferred_element_type=F32)
    xl2 = c2_ref[:, :HID]
    xr2 = c2_ref[:, HID:]
    mm = xl2 + xr2 + lp2
    ml = jnp.maximum(mm, 0.2 * mm)
    al = jnp.dot(ml, att2_ref[...], preferred_element_type=F32)
    ex = jnp.exp(al)
    h2 = (accs[:, :HID] + ex * xl2) / (den + ex + 1e-16) + bias_ref[...]
    h2 = jnp.where(h2 > 0, h2, jnp.exp(jnp.minimum(h2, 0.0)) - 1.0)
    g_ref[...] = jnp.concatenate(
        [jnp.dot(h2, wsT_ref[...], preferred_element_type=F32),
         jnp.dot(h2, wdT_ref[...], preferred_element_type=F32)], axis=1)


def _ep2(acc, c2, la, ew2T, att2, s64, bias, wsT, wdT):
    def full(shape):
        return pl.BlockSpec(shape, lambda i, _n=len(shape): (0,) * _n)
    return pl.pallas_call(
        _ep2_body,
        grid=(N // BN,),
        in_specs=[
            pl.BlockSpec((NC, BN, 128), lambda i: (0, i, 0)),
            pl.BlockSpec((BN, 128), lambda i: (i, 0)),
            pl.BlockSpec((BN, ED), lambda i: (i, 0)),
            full((ED, HID)),
            full((HID, 1)),
            full((128, 1)),
            full((1, HID)),
            full((HID, HID)),
            full((HID, HID)),
        ],
        out_specs=pl.BlockSpec((BN, 128), lambda i: (i, 0)),
        out_shape=jax.ShapeDtypeStruct((N, 128), F32),
    )(acc, c2, la, ew2T, att2, s64, bias, wsT, wdT)


_S16 = np.zeros((128, 1), np.float32)
_S16[16, 0] = 1.0
_S64 = np.zeros((128, 1), np.float32)
_S64[64, 0] = 1.0


def kernel(x, edge_index, edge_attr, l1_lw, l1_lb, l1_rw, l1_rb, l1_ew,
           l1_att, l1_bias, l2_lw, l2_lb, l2_rw, l2_rb, l2_ew, l2_att,
           l2_bias, c1_w, c1_b, c2_w, c2_b):
    src = edge_index[0]
    dst = edge_index[1]
    eae = edge_attr[0::2]
    eao = edge_attr[1::2]
    eaf = edge_attr.reshape(-1)

    def hsplit(w):
        return w.reshape(w.shape[0], 4, HID).transpose(1, 0, 2)

    tbl1 = _t1(x, hsplit(l1_lw.T), l1_lb.reshape(4, 1, HID),
               hsplit(l1_rw.T), l1_rb.reshape(4, 1, HID))
    ep1 = _t2a(eae, eao, hsplit(l1_ew.T))
    ep2, eec = _t2b(eae, eao, l2_ew.T, c1_w[:, 128:].T, c1_b[None])

    s0acc = _s0(dst, eaf)

    tblf = tbl1.reshape(4 * N, 128)
    epf = ep1.reshape(4 * (E // 2), 128)
    attm1 = l1_att.reshape(-1)
    acca = _s1h(0, src, dst, tblf, epf, attm1)
    accb = _s1h(1, src, dst, tblf, epf, attm1)

    attmask = jnp.zeros((256, 4), F32).at[
        jnp.arange(256), jnp.arange(256) // 64].set(l1_att.reshape(-1))
    c2tbl, la = _ep1(
        acca.reshape(NC, N, 128), accb.reshape(NC, N, 128),
        s0acc.reshape(NC, N, 128), tbl1,
        l1_ew.T, jnp.asarray(_S16), jnp.asarray(_S64),
        attmask, l1_bias[None],
        l2_lw.T, l2_lb[None], l2_rw.T, l2_rb[None])

    acc2 = _s2(src, dst, c2tbl, ep2, l2_att.reshape(-1))

    gtbl = _ep2(acc2.reshape(NC, N, 128), c2tbl, la,
                l2_ew.T, l2_att.reshape(HID, 1), jnp.asarray(_S64),
                l2_bias[None], c1_w[:, :64].T, c1_w[:, 64:128].T)

    cwm = jnp.concatenate([c2_w.reshape(-1),
                           jnp.broadcast_to(c2_b, (16,))])
    logits = _s3(src, dst, gtbl, eec, cwm)
    return logits


# speedup vs baseline: 12.5032x; 1.2076x over previous
import functools

import jax
import jax.numpy as jnp
import numpy as np
from jax import lax
from jax.experimental import pallas as pl
from jax.experimental.pallas import tpu as pltpu
from jax.experimental.pallas import tpu_sc as plsc

N = 10000
E = 320000
ND = 128
ED = 16
HID = 64
NC = 2
NS = 16
CB = 80
CB1 = 40
BN = 1000
BEP = 2000

F32 = jnp.float32
I32 = jnp.int32

_MESH = dict(core_axis_name="c", subcore_axis_name="s")

_EPW = E // (NC * NS)
_EPT = E // NS
_SEG = 624
_SEGC = 104
_TAIL = N - NS * _SEG

_DNUMS = lax.GatherDimensionNumbers(
    offset_dims=(), collapsed_slice_dims=(0,), start_index_map=(0,))


def _allsum(v):
    lanes = lax.iota(I32, 16)
    for k in range(4):
        p = lax.bitwise_xor(lanes, 1 << k)
        v = v + lax.gather(v, p[:, None], _DNUMS, slice_sizes=(1,),
                           mode=lax.GatherScatterMode.PROMISE_IN_BOUNDS)
    return v


def _zeros16():
    return jnp.zeros((16,), F32)


def _m8(v):
    return pl.multiple_of(v, 8)


def _zero_vmem(buf, w):
    z16 = _zeros16()

    def zr(i, _):
        for q in range(w // 16):
            buf[i, pl.ds(q * 16, 16)] = z16
        return 0
    lax.fori_loop(0, buf.shape[0], zr, 0)


def _zero_shared(zb, sh, sid):
    for t in range(_SEG // _SEGC):
        pltpu.sync_copy(zb, sh.at[pl.ds(_m8(sid * _SEG + t * _SEGC), _SEGC)])

    @pl.when(sid == NS - 1)
    def _():
        pltpu.sync_copy(zb.at[pl.ds(0, _TAIL)], sh.at[pl.ds(NS * _SEG, _TAIL)])


def _copy_out_shared(sh, hbm, cid, sid):
    pltpu.sync_copy(sh.at[pl.ds(_m8(sid * _SEG), _SEG)],
                    hbm.at[pl.ds(_m8(cid * N + sid * _SEG), _SEG)])

    @pl.when(sid == NS - 1)
    def _():
        pltpu.sync_copy(sh.at[pl.ds(NS * _SEG, _TAIL)],
                        hbm.at[pl.ds(_m8(cid * N + NS * _SEG), _TAIL)])


def _t1_body(x_ref, lwT_ref, lb_ref, rwT_ref, rb_ref, c_ref):
    xv = x_ref[...]
    xl = jnp.dot(xv, lwT_ref[0], preferred_element_type=F32) + lb_ref[0]
    xr = jnp.dot(xv, rwT_ref[0], preferred_element_type=F32) + rb_ref[0]
    c_ref[...] = jnp.concatenate([xl, xr], axis=1)[None]


def _t1(x, lwT, lb, rwT, rb):
    return pl.pallas_call(
        _t1_body,
        grid=(4, N // BN),
        in_specs=[
            pl.BlockSpec((BN, ND), lambda c, i: (i, 0)),
            pl.BlockSpec((1, ND, HID), lambda c, i: (c, 0, 0)),
            pl.BlockSpec((1, 1, HID), lambda c, i: (c, 0, 0)),
            pl.BlockSpec((1, ND, HID), lambda c, i: (c, 0, 0)),
            pl.BlockSpec((1, 1, HID), lambda c, i: (c, 0, 0)),
        ],
        out_specs=pl.BlockSpec((1, BN, 128), lambda c, i: (c, i, 0)),
        out_shape=jax.ShapeDtypeStruct((4, N, 128), F32),
    )(x, lwT, lb, rwT, rb)


def _t2a_body(eae_ref, eao_ref, ewT_ref, ep_ref):
    pe = jnp.dot(eae_ref[...], ewT_ref[0], preferred_element_type=F32)
    po = jnp.dot(eao_ref[...], ewT_ref[0], preferred_element_type=F32)
    ep_ref[...] = jnp.concatenate([pe, po], axis=1)[None]


def _t2a(eae, eao, ewT):
    return pl.pallas_call(
        _t2a_body,
        grid=(4, (E // 2) // BEP),
        in_specs=[
            pl.BlockSpec((BEP, ED), lambda c, i: (i, 0)),
            pl.BlockSpec((BEP, ED), lambda c, i: (i, 0)),
            pl.BlockSpec((1, ED, HID), lambda c, i: (c, 0, 0)),
        ],
        out_specs=pl.BlockSpec((1, BEP, 128), lambda c, i: (c, i, 0)),
        out_shape=jax.ShapeDtypeStruct((4, E // 2, 128), F32),
    )(eae, eao, ewT)


def _t2b_body(eae_ref, eao_ref, w2T_ref, weT_ref, cb_ref, ep2_ref, eec_ref):
    eav = eae_ref[...]
    eov = eao_ref[...]
    ep2_ref[...] = jnp.concatenate(
        [jnp.dot(eav, w2T_ref[...], preferred_element_type=F32),
         jnp.dot(eov, w2T_ref[...], preferred_element_type=F32)], axis=1)
    eec_ref[...] = jnp.concatenate(
        [jnp.dot(eav, weT_ref[...], preferred_element_type=F32) + cb_ref[...],
         jnp.dot(eov, weT_ref[...], preferred_element_type=F32) + cb_ref[...]],
        axis=1)


def _t2b(eae, eao, w2T, weT, cb):
    return pl.pallas_call(
        _t2b_body,
        grid=((E // 2) // BEP,),
        in_specs=[
            pl.BlockSpec((BEP, ED), lambda i: (i, 0)),
            pl.BlockSpec((BEP, ED), lambda i: (i, 0)),
            pl.BlockSpec((ED, HID), lambda i: (0, 0)),
            pl.BlockSpec((ED, HID), lambda i: (0, 0)),
            pl.BlockSpec((1, HID), lambda i: (0, 0)),
        ],
        out_specs=[pl.BlockSpec((BEP, 128), lambda i: (i, 0))] * 2,
        out_shape=[jax.ShapeDtypeStruct((E // 2, 128), F32)] * 2,
    )(eae, eao, w2T, weT, cb)


def _s0_body(dst_hbm, ea_hbm, acc_hbm,
             dstv, eab, wbuf, zb, acc_sh, sem):
    cid = lax.axis_index("c")
    sid = lax.axis_index("s")
    wid = cid * NS + sid
    lanes = lax.iota(I32, 16)
    mask0 = jnp.where(lanes == 0, 1.0, 0.0).astype(F32)

    _zero_vmem(wbuf, 128)

    def fill(i, _):
        wbuf[i, pl.ds(16, 16)] = mask0
        return 0
    lax.fori_loop(0, CB, fill, 0)
    _zero_vmem(zb, 128)
    _zero_shared(zb, acc_sh, sid)
    plsc.subcore_barrier()

    def chunk(k, _):
        base = _m8(wid * _EPW + k * CB)
        pltpu.sync_copy(dst_hbm.at[pl.ds(base, CB)], dstv)
        pltpu.sync_copy(ea_hbm.at[pl.ds(_m8(base * ED), CB * ED)], eab)

        def edge(j, _):
            wbuf[j, pl.ds(0, 16)] = eab[pl.ds(j * ED, 16)]
            return 0
        lax.fori_loop(0, CB, edge, 0)
        pltpu.sync_copy(wbuf, acc_sh.at[dstv], add=True)
        return 0
    lax.fori_loop(0, _EPW // CB, chunk, 0)
    plsc.subcore_barrier()

    _copy_out_shared(acc_sh, acc_hbm, cid, sid)


def _s0(dst, eaf):
    return pl.kernel(
        _s0_body,
        out_type=jax.ShapeDtypeStruct((NC * N, 128), F32),
        mesh=plsc.VectorSubcoreMesh(**_MESH),
        scratch_types=[
            pltpu.VMEM((CB,), I32),
            pltpu.VMEM((CB * ED,), F32),
            pltpu.VMEM((CB, 128), F32),
            pltpu.VMEM((_SEGC, 128), F32),
            pltpu.VMEM_SHARED((N, 128), F32),
            pltpu.SemaphoreType.DMA,
        ],
    )(dst, eaf)


def _s1h_body(t, soff_hbm, doff_hbm, dstr_hbm, tbl_hbm, ep_hbm, att_hbm,
              acc_hbm,
              si0, si1, di0, di1, dr0, dr1, drs0, drs1,
              srow0, srow1, drow0, drow1, epb, wbuf0, wbuf1,
              attb, zb, acc_sh,
              semi0, semi1, semg0, semg1, sems0, sems1, semp):
    cid = lax.axis_index("c")
    sid = lax.axis_index("s")
    lanes = lax.iota(I32, 16)
    mask0 = jnp.where(lanes == 0, 1.0, 0.0).astype(F32)
    head = 2 * t + cid

    si = (si0, si1)
    di = (di0, di1)
    dr = (dr0, dr1)
    drs = (drs0, drs1)
    srow = (srow0, srow1)
    drow = (drow0, drow1)
    wbuf = (wbuf0, wbuf1)
    semi = (semi0, semi1)
    semg = (semg0, semg1)
    sems = (sems0, sems1)

    pltpu.sync_copy(att_hbm, attb)
    att_vecs = [attb[pl.ds(head * HID + q * 16, 16)] for q in range(4)]

    for b in range(2):
        _zero_vmem(wbuf[b], 128)
    _zero_vmem(zb, 128)
    z16i = jnp.zeros((16,), I32)
    for b in range(2):
        for g in range(CB1 // 16 + 1):
            drs[b][pl.ds(min(g * 16, CB1 - 16), 16)] = z16i
    _zero_shared(zb, acc_sh, sid)
    plsc.subcore_barrier()

    nch = _EPT // CB1
    ebase = head * E + sid * _EPT
    rbase = sid * _EPT

    def fire_idx(c, b):
        bs = _m8(ebase + c * CB1)
        br = _m8(rbase + c * CB1)
        pltpu.async_copy(soff_hbm.at[pl.ds(bs, CB1)], si[b], semi[b])
        pltpu.async_copy(doff_hbm.at[pl.ds(bs, CB1)], di[b], semi[b])
        pltpu.async_copy(dstr_hbm.at[pl.ds(br, CB1)], dr[b], semi[b])

    def wait_idx(b):
        pltpu.make_async_copy(soff_hbm.at[pl.ds(0, CB1)], si[b], semi[b]).wait()
        pltpu.make_async_copy(doff_hbm.at[pl.ds(0, CB1)], di[b], semi[b]).wait()
        pltpu.make_async_copy(dstr_hbm.at[pl.ds(0, CB1)], dr[b], semi[b]).wait()

    def fire_gather(c, b):
        pltpu.async_copy(tbl_hbm.at[si[b]], srow[b], semg[b])
        pltpu.async_copy(tbl_hbm.at[di[b]], drow[b], semg[b])

    def wait_gather(b):
        pltpu.make_async_copy(tbl_hbm.at[si[b]], srow[b], semg[b]).wait()
        pltpu.make_async_copy(tbl_hbm.at[di[b]], drow[b], semg[b]).wait()

    def wait_scatter(b):
        pltpu.make_async_copy(acc_hbm.at[pl.ds(0, CB1)], wbuf[b],
                              sems[b]).wait()

    for b in range(2):
        pltpu.async_copy(wbuf[b], acc_sh.at[drs[b]], sems[b], add=True)
        fire_idx(b, b)
    for b in range(2):
        wait_idx(b)
        fire_gather(b, b)

    def body(k2, _):
        bp = _m8((ebase + 2 * k2 * CB1) // 2)
        cpe = pltpu.async_copy(ep_hbm.at[pl.ds(bp, CB1)], epb, semp)
        for b in range(2):
            c = 2 * k2 + b
            wait_scatter(b)
            wait_gather(b)
            if b == 0:
                cpe.wait()
            for g in range(CB1 // 16 + 1):
                sl = pl.ds(min(g * 16, CB1 - 16), 16)
                drs[b][sl] = dr[b][sl]
            cn = c + 2
            cc = jnp.where(cn < nch, cn, 0)
            fire_idx(cc, b)
            sr = srow[b]
            drr = drow[b]
            wb = wbuf[b]
            prow = b * (CB1 // 2)

            def pair(p, _):
                for u in range(2):
                    j = 2 * p + u
                    acc = _zeros16()
                    for q in range(4):
                        s = (sr[j, pl.ds(q * 16, 16)]
                             + drr[j, pl.ds(64 + q * 16, 16)]
                             + epb[prow + p, pl.ds(u * 64 + q * 16, 16)])
                        m = jnp.maximum(s, 0.2 * s)
                        acc = acc + m * att_vecs[q]
                    ex = jnp.exp(_allsum(acc))
                    for q in range(4):
                        sl = pl.ds(q * 16, 16)
                        wb[j, sl] = sr[j, sl] * ex
                    wb[j, pl.ds(64, 16)] = ex * mask0
                return 0
            lax.fori_loop(0, CB1 // 2, pair, 0)
            pltpu.async_copy(wb, acc_sh.at[drs[b]], sems[b], add=True)
            wait_idx(b)
            fire_gather(cc, b)
        return 0
    lax.fori_loop(0, nch // 2, body, 0)
    for b in range(2):
        wait_scatter(b)
        wait_gather(b)
    plsc.subcore_barrier()

    _copy_out_shared(acc_sh, acc_hbm, cid, sid)


def _s1h(t, soff, doff, dstr, tbl, ep, attf):
    return pl.kernel(
        functools.partial(_s1h_body, t),
        out_type=jax.ShapeDtypeStruct((NC * N, 128), F32),
        mesh=plsc.VectorSubcoreMesh(**_MESH),
        scratch_types=[
            pltpu.VMEM((CB1,), I32),
            pltpu.VMEM((CB1,), I32),
            pltpu.VMEM((CB1,), I32),
            pltpu.VMEM((CB1,), I32),
            pltpu.VMEM((CB1,), I32),
            pltpu.VMEM((CB1,), I32),
            pltpu.VMEM((CB1,), I32),
            pltpu.VMEM((CB1,), I32),
            pltpu.VMEM((CB1, 128), F32),
            pltpu.VMEM((CB1, 128), F32),
            pltpu.VMEM((CB1, 128), F32),
            pltpu.VMEM((CB1, 128), F32),
            pltpu.VMEM((CB1, 128), F32),
            pltpu.VMEM((CB1, 128), F32),
            pltpu.VMEM((CB1, 128), F32),
            pltpu.VMEM((256,), F32),
            pltpu.VMEM((_SEGC, 128), F32),
            pltpu.VMEM_SHARED((N, 128), F32),
            pltpu.SemaphoreType.DMA,
            pltpu.SemaphoreType.DMA,
            pltpu.SemaphoreType.DMA,
            pltpu.SemaphoreType.DMA,
            pltpu.SemaphoreType.DMA,
            pltpu.SemaphoreType.DMA,
            pltpu.SemaphoreType.DMA,
        ],
    )(soff, doff, dstr, tbl, ep, attf)


def _s2_body(src_hbm, dst_hbm, tbl_hbm, ep_hbm, att_hbm,
             acc_hbm,
             idxl, idxr, srow, drow, epb, wbuf, attb, zb,
             acc_sh, sem):
    cid = lax.axis_index("c")
    sid = lax.axis_index("s")
    wid = cid * NS + sid
    lanes = lax.iota(I32, 16)
    mask0 = jnp.where(lanes == 0, 1.0, 0.0).astype(F32)

    pltpu.sync_copy(att_hbm, attb)
    att_vecs = [attb[pl.ds(q * 16, 16)] for q in range(4)]

    _zero_vmem(wbuf, 128)
    _zero_vmem(zb, 128)
    _zero_shared(zb, acc_sh, sid)
    plsc.subcore_barrier()

    def chunk(k, _):
        base = _m8(wid * _EPW + k * CB)
        pltpu.sync_copy(src_hbm.at[pl.ds(base, CB)], idxl)
        pltpu.sync_copy(dst_hbm.at[pl.ds(base, CB)], idxr)
        cpl = pltpu.async_copy(tbl_hbm.at[idxl], srow, sem)
        cpr = pltpu.async_copy(tbl_hbm.at[idxr], drow, sem)
        cpe = pltpu.async_copy(ep_hbm.at[pl.ds(_m8(base // 2), CB // 2)], epb, sem)
        cpl.wait()
        cpr.wait()
        cpe.wait()

        def pair(p, _):
            for u in range(2):
                j = 2 * p + u
                acc = _zeros16()
                for q in range(4):
                    s = (srow[j, pl.ds(q * 16, 16)]
                         + drow[j, pl.ds(64 + q * 16, 16)]
                         + epb[p, pl.ds(u * 64 + q * 16, 16)])
                    m = jnp.maximum(s, 0.2 * s)
                    acc = acc + m * att_vecs[q]
                ex = jnp.exp(_allsum(acc))
                for q in range(4):
                    sl = pl.ds(q * 16, 16)
                    wbuf[j, sl] = srow[j, sl] * ex
                wbuf[j, pl.ds(64, 16)] = ex * mask0
            return 0
        lax.fori_loop(0, CB // 2, pair, 0)
        pltpu.sync_copy(wbuf, acc_sh.at[idxr], add=True)
        return 0
    lax.fori_loop(0, _EPW // CB, chunk, 0)
    plsc.subcore_barrier()

    _copy_out_shared(acc_sh, acc_hbm, cid, sid)


def _s2(src, dst, tbl, ep, attf):
    return pl.kernel(
        _s2_body,
        out_type=jax.ShapeDtypeStruct((NC * N, 128), F32),
        mesh=plsc.VectorSubcoreMesh(**_MESH),
        scratch_types=[
            pltpu.VMEM((CB,), I32),
            pltpu.VMEM((CB,), I32),
            pltpu.VMEM((CB, 128), F32),
            pltpu.VMEM((CB, 128), F32),
            pltpu.VMEM((CB // 2, 128), F32),
            pltpu.VMEM((CB, 128), F32),
            pltpu.VMEM((64,), F32),
            pltpu.VMEM((_SEGC, 128), F32),
            pltpu.VMEM_SHARED((N, 128), F32),
            pltpu.SemaphoreType.DMA,
        ],
    )(src, dst, tbl, ep, attf)


def _s3_body(src_hbm, dst_hbm, g_hbm, eec_hbm, cw_hbm,
             lg_hbm,
             idxl, idxr, srow, drow, ecb, wb, lbuf, sem):
    cid = lax.axis_index("c")
    sid = lax.axis_index("s")
    wid = cid * NS + sid
    lanes = lax.iota(I32, 16)

    pltpu.sync_copy(cw_hbm, wb)
    w_vecs = [wb[pl.ds(q * 16, 16)] for q in range(4)]
    cb_vec = wb[pl.ds(64, 16)]

    def chunk(k, _):
        base = _m8(wid * _EPW + k * CB)
        pltpu.sync_copy(src_hbm.at[pl.ds(base, CB)], idxl)
        pltpu.sync_copy(dst_hbm.at[pl.ds(base, CB)], idxr)
        cpl = pltpu.async_copy(g_hbm.at[idxl], srow, sem)
        cpr = pltpu.async_copy(g_hbm.at[idxr], drow, sem)
        cpe = pltpu.async_copy(eec_hbm.at[pl.ds(_m8(base // 2), CB // 2)], ecb, sem)
        cpl.wait()
        cpr.wait()
        cpe.wait()

        def group(g, _):
            lvec = cb_vec
            for jj in range(16):
                j = g * 16 + jj
                acc = _zeros16()
                for q in range(4):
                    z = (srow[j, pl.ds(q * 16, 16)]
                         + drow[j, pl.ds(64 + q * 16, 16)]
                         + ecb[(g * 16 + jj) // 2,
                               pl.ds((jj % 2) * 64 + q * 16, 16)])
                    z = jnp.maximum(z, 0.0)
                    acc = acc + z * w_vecs[q]
                a = _allsum(acc)
                mj = jnp.where(lanes == jj, 1.0, 0.0).astype(F32)
                lvec = lvec + a * mj
            lbuf[pl.ds(g * 16, 16)] = lvec
            return 0
        lax.fori_loop(0, CB // 16, group, 0)
        pltpu.sync_copy(lbuf, lg_hbm.at[pl.ds(base, CB)])
        return 0
    lax.fori_loop(0, _EPW // CB, chunk, 0)


def _s3(src, dst, g, eec, cwm):
    return pl.kernel(
        _s3_body,
        out_type=jax.ShapeDtypeStruct((E,), F32),
        mesh=plsc.VectorSubcoreMesh(**_MESH),
        scratch_types=[
            pltpu.VMEM((CB,), I32),
            pltpu.VMEM((CB,), I32),
            pltpu.VMEM((CB, 128), F32),
            pltpu.VMEM((CB, 128), F32),
            pltpu.VMEM((CB // 2, 128), F32),
            pltpu.VMEM((80,), F32),
            pltpu.VMEM((CB,), F32),
            pltpu.SemaphoreType.DMA,
        ],
    )(src, dst, g, eec, cwm)


def _ep1_body(acca_ref, accb_ref, s0_ref, xlr_ref,
              ewT_ref, s16_ref, s64_ref, attm_ref, bias_ref,
              l2lwT_ref, l2lb_ref, l2rwT_ref, l2rb_ref,
              c2_ref, la_ref):
    s0s = s0_ref[0] + s0_ref[1]
    ea = s0s[:, :ED]
    cntv = jnp.dot(s0s, s16_ref[...], preferred_element_type=F32)
    la = ea / jnp.maximum(cntv, 1.0)
    la_ref[...] = la
    lp = jnp.dot(la, ewT_ref[...], preferred_element_type=F32)

    accs = [acca_ref[0], acca_ref[1], accb_ref[0], accb_ref[1]]
    xls = []
    mls = []
    for h in range(4):
        xlh = xlr_ref[h][:, :HID]
        xrh = xlr_ref[h][:, HID:]
        mm = xlh + xrh + lp[:, h * HID:(h + 1) * HID]
        xls.append(xlh)
        mls.append(jnp.maximum(mm, 0.2 * mm))
    al = sum(jnp.dot(mls[h], attm_ref[h * HID:(h + 1) * HID, :],
                     preferred_element_type=F32) for h in range(4))
    exl = jnp.exp(al)
    xl2v = l2lb_ref[...]
    xr2v = l2rb_ref[...]
    for h in range(4):
        exh = exl[:, h:h + 1]
        denh = jnp.dot(accs[h], s64_ref[...], preferred_element_type=F32)
        num = accs[h][:, :HID] + exh * xls[h]
        hv = num / (denh + exh + 1e-16) + bias_ref[:, h * HID:(h + 1) * HID]
        hv = jnp.where(hv > 0, hv, jnp.exp(jnp.minimum(hv, 0.0)) - 1.0)
        xl2v = xl2v + jnp.dot(hv, l2lwT_ref[h * HID:(h + 1) * HID, :],
                              preferred_element_type=F32)
        xr2v = xr2v + jnp.dot(hv, l2rwT_ref[h * HID:(h + 1) * HID, :],
                              preferred_element_type=F32)
    c2_ref[...] = jnp.concatenate([xl2v, xr2v], axis=1)


def _ep1(acca, accb, s0acc, xlr, ewT, s16, s64, attm, bias,
         l2lwT, l2lb, l2rwT, l2rb):
    def full(shape):
        return pl.BlockSpec(shape, lambda i, _n=len(shape): (0,) * _n)
    return pl.pallas_call(
        _ep1_body,
        grid=(N // BN,),
        in_specs=[
            pl.BlockSpec((NC, BN, 128), lambda i: (0, i, 0)),
            pl.BlockSpec((NC, BN, 128), lambda i: (0, i, 0)),
            pl.BlockSpec((NC, BN, 128), lambda i: (0, i, 0)),
            pl.BlockSpec((4, BN, 128), lambda i: (0, i, 0)),
            full((ED, 256)),
            full((128, 1)),
            full((128, 1)),
            full((256, 4)),
            full((1, 256)),
            full((256, HID)),
            full((1, HID)),
            full((256, HID)),
            full((1, HID)),
        ],
        out_specs=[pl.BlockSpec((BN, 128), lambda i: (i, 0)),
                   pl.BlockSpec((BN, ED), lambda i: (i, 0))],
        out_shape=[jax.ShapeDtypeStruct((N, 128), F32),
                   jax.ShapeDtypeStruct((N, ED), F32)],
    )(acca, accb, s0acc, xlr, ewT, s16, s64, attm, bias,
      l2lwT, l2lb, l2rwT, l2rb)


def _ep2_body(acc_ref, c2_ref, la_ref,
              ew2T_ref, att2_ref, s64_ref, bias_ref, wsT_ref, wdT_ref,
              g_ref):
    accs = acc_ref[0] + acc_ref[1]
    den = jnp.dot(accs, s64_ref[...], preferred_element_type=F32)
    lp2 = jnp.dot(la_ref[...], ew2T_ref[...], preferred_element_type=F32)
    xl2 = c2_ref[:, :HID]
    xr2 = c2_ref[:, HID:]
    mm = xl2 + xr2 + lp2
    ml = jnp.maximum(mm, 0.2 * mm)
    al = jnp.dot(ml, att2_ref[...], preferred_element_type=F32)
    ex = jnp.exp(al)
    h2 = (accs[:, :HID] + ex * xl2) / (den + ex + 1e-16) + bias_ref[...]
    h2 = jnp.where(h2 > 0, h2, jnp.exp(jnp.minimum(h2, 0.0)) - 1.0)
    g_ref[...] = jnp.concatenate(
        [jnp.dot(h2, wsT_ref[...], preferred_element_type=F32),
         jnp.dot(h2, wdT_ref[...], preferred_element_type=F32)], axis=1)


def _ep2(acc, c2, la, ew2T, att2, s64, bias, wsT, wdT):
    def full(shape):
        return pl.BlockSpec(shape, lambda i, _n=len(shape): (0,) * _n)
    return pl.pallas_call(
        _ep2_body,
        grid=(N // BN,),
        in_specs=[
            pl.BlockSpec((NC, BN, 128), lambda i: (0, i, 0)),
            pl.BlockSpec((BN, 128), lambda i: (i, 0)),
            pl.BlockSpec((BN, ED), lambda i: (i, 0)),
            full((ED, HID)),
            full((HID, 1)),
            full((128, 1)),
            full((1, HID)),
            full((HID, HID)),
            full((HID, HID)),
        ],
        out_specs=pl.BlockSpec((BN, 128), lambda i: (i, 0)),
        out_shape=jax.ShapeDtypeStruct((N, 128), F32),
    )(acc, c2, la, ew2T, att2, s64, bias, wsT, wdT)


_S16 = np.zeros((128, 1), np.float32)
_S16[16, 0] = 1.0
_S64 = np.zeros((128, 1), np.float32)
_S64[64, 0] = 1.0


def kernel(x, edge_index, edge_attr, l1_lw, l1_lb, l1_rw, l1_rb, l1_ew,
           l1_att, l1_bias, l2_lw, l2_lb, l2_rw, l2_rb, l2_ew, l2_att,
           l2_bias, c1_w, c1_b, c2_w, c2_b):
    src = edge_index[0]
    dst = edge_index[1]
    eae = edge_attr[0::2]
    eao = edge_attr[1::2]
    eaf = edge_attr.reshape(-1)

    def hsplit(w):
        return w.reshape(w.shape[0], 4, HID).transpose(1, 0, 2)

    tbl1 = _t1(x, hsplit(l1_lw.T), l1_lb.reshape(4, 1, HID),
               hsplit(l1_rw.T), l1_rb.reshape(4, 1, HID))
    ep1 = _t2a(eae, eao, hsplit(l1_ew.T))
    ep2, eec = _t2b(eae, eao, l2_ew.T, c1_w[:, 128:].T, c1_b[None])

    s0acc = _s0(dst, eaf)

    tblf = tbl1.reshape(4 * N, 128)
    epf = ep1.reshape(4 * (E // 2), 128)
    attm1 = l1_att.reshape(-1)
    hoff = (jnp.arange(4, dtype=I32) * N)[:, None]
    srcoff = (src[None, :] + hoff).reshape(-1)
    dstoff = (dst[None, :] + hoff).reshape(-1)
    acca = _s1h(0, srcoff, dstoff, dst, tblf, epf, attm1)
    accb = _s1h(1, srcoff, dstoff, dst, tblf, epf, attm1)

    attmask = jnp.zeros((256, 4), F32).at[
        jnp.arange(256), jnp.arange(256) // 64].set(l1_att.reshape(-1))
    c2tbl, la = _ep1(
        acca.reshape(NC, N, 128), accb.reshape(NC, N, 128),
        s0acc.reshape(NC, N, 128), tbl1,
        l1_ew.T, jnp.asarray(_S16), jnp.asarray(_S64),
        attmask, l1_bias[None],
        l2_lw.T, l2_lb[None], l2_rw.T, l2_rb[None])

    acc2 = _s2(src, dst, c2tbl, ep2, l2_att.reshape(-1))

    gtbl = _ep2(acc2.reshape(NC, N, 128), c2tbl, la,
                l2_ew.T, l2_att.reshape(HID, 1), jnp.asarray(_S64),
                l2_bias[None], c1_w[:, :64].T, c1_w[:, 64:128].T)

    cwm = jnp.concatenate([c2_w.reshape(-1),
                           jnp.broadcast_to(c2_b, (16,))])
    logits = _s3(src, dst, gtbl, eec, cwm)
    return logits


# speedup vs baseline: 13.5300x; 1.0821x over previous
import functools

import jax
import jax.numpy as jnp
import numpy as np
from jax import lax
from jax.experimental import pallas as pl
from jax.experimental.pallas import tpu as pltpu
from jax.experimental.pallas import tpu_sc as plsc

N = 10000
E = 320000
ND = 128
ED = 16
HID = 64
NC = 2
NS = 16
CB = 80
CB1 = 40
BN = 1000
BEP = 2000

F32 = jnp.float32
I32 = jnp.int32

_MESH = dict(core_axis_name="c", subcore_axis_name="s")

_EPW = E // (NC * NS)
_EPT = E // NS
_SEG = 624
_SEGC = 104
_TAIL = N - NS * _SEG

_DNUMS = lax.GatherDimensionNumbers(
    offset_dims=(), collapsed_slice_dims=(0,), start_index_map=(0,))


def _allsum(v):
    lanes = lax.iota(I32, 16)
    for k in range(4):
        p = lax.bitwise_xor(lanes, 1 << k)
        v = v + lax.gather(v, p[:, None], _DNUMS, slice_sizes=(1,),
                           mode=lax.GatherScatterMode.PROMISE_IN_BOUNDS)
    return v


def _zeros16():
    return jnp.zeros((16,), F32)


def _m8(v):
    return pl.multiple_of(v, 8)


def _zero_vmem(buf, w):
    z16 = _zeros16()

    def zr(i, _):
        for q in range(w // 16):
            buf[i, pl.ds(q * 16, 16)] = z16
        return 0
    lax.fori_loop(0, buf.shape[0], zr, 0)


def _zero_shared(zb, sh, sid):
    for t in range(_SEG // _SEGC):
        pltpu.sync_copy(zb, sh.at[pl.ds(_m8(sid * _SEG + t * _SEGC), _SEGC)])

    @pl.when(sid == NS - 1)
    def _():
        pltpu.sync_copy(zb.at[pl.ds(0, _TAIL)], sh.at[pl.ds(NS * _SEG, _TAIL)])


def _copy_out_shared(sh, hbm, cid, sid):
    pltpu.sync_copy(sh.at[pl.ds(_m8(sid * _SEG), _SEG)],
                    hbm.at[pl.ds(_m8(cid * N + sid * _SEG), _SEG)])

    @pl.when(sid == NS - 1)
    def _():
        pltpu.sync_copy(sh.at[pl.ds(NS * _SEG, _TAIL)],
                        hbm.at[pl.ds(_m8(cid * N + NS * _SEG), _TAIL)])


def _t1_body(x_ref, lwT_ref, lb_ref, rwT_ref, rb_ref, c_ref):
    xv = x_ref[...]
    xl = jnp.dot(xv, lwT_ref[0], preferred_element_type=F32) + lb_ref[0]
    xr = jnp.dot(xv, rwT_ref[0], preferred_element_type=F32) + rb_ref[0]
    c_ref[...] = jnp.concatenate([xl, xr], axis=1)[None]


def _t1(x, lwT, lb, rwT, rb):
    return pl.pallas_call(
        _t1_body,
        grid=(4, N // BN),
        in_specs=[
            pl.BlockSpec((BN, ND), lambda c, i: (i, 0)),
            pl.BlockSpec((1, ND, HID), lambda c, i: (c, 0, 0)),
            pl.BlockSpec((1, 1, HID), lambda c, i: (c, 0, 0)),
            pl.BlockSpec((1, ND, HID), lambda c, i: (c, 0, 0)),
            pl.BlockSpec((1, 1, HID), lambda c, i: (c, 0, 0)),
        ],
        out_specs=pl.BlockSpec((1, BN, 128), lambda c, i: (c, i, 0)),
        out_shape=jax.ShapeDtypeStruct((4, N, 128), F32),
    )(x, lwT, lb, rwT, rb)


def _t2a_body(eae_ref, eao_ref, ewT_ref, ep_ref):
    pe = jnp.dot(eae_ref[...], ewT_ref[0], preferred_element_type=F32)
    po = jnp.dot(eao_ref[...], ewT_ref[0], preferred_element_type=F32)
    ep_ref[...] = jnp.concatenate([pe, po], axis=1)[None]


def _t2a(eae, eao, ewT):
    return pl.pallas_call(
        _t2a_body,
        grid=(4, (E // 2) // BEP),
        in_specs=[
            pl.BlockSpec((BEP, ED), lambda c, i: (i, 0)),
            pl.BlockSpec((BEP, ED), lambda c, i: (i, 0)),
            pl.BlockSpec((1, ED, HID), lambda c, i: (c, 0, 0)),
        ],
        out_specs=pl.BlockSpec((1, BEP, 128), lambda c, i: (c, i, 0)),
        out_shape=jax.ShapeDtypeStruct((4, E // 2, 128), F32),
    )(eae, eao, ewT)


def _t2b_body(eae_ref, eao_ref, w2T_ref, weT_ref, cb_ref, ep2_ref, eec_ref):
    eav = eae_ref[...]
    eov = eao_ref[...]
    ep2_ref[...] = jnp.concatenate(
        [jnp.dot(eav, w2T_ref[...], preferred_element_type=F32),
         jnp.dot(eov, w2T_ref[...], preferred_element_type=F32)], axis=1)
    eec_ref[...] = jnp.concatenate(
        [jnp.dot(eav, weT_ref[...], preferred_element_type=F32) + cb_ref[...],
         jnp.dot(eov, weT_ref[...], preferred_element_type=F32) + cb_ref[...]],
        axis=1)


def _t2b(eae, eao, w2T, weT, cb):
    return pl.pallas_call(
        _t2b_body,
        grid=((E // 2) // BEP,),
        in_specs=[
            pl.BlockSpec((BEP, ED), lambda i: (i, 0)),
            pl.BlockSpec((BEP, ED), lambda i: (i, 0)),
            pl.BlockSpec((ED, HID), lambda i: (0, 0)),
            pl.BlockSpec((ED, HID), lambda i: (0, 0)),
            pl.BlockSpec((1, HID), lambda i: (0, 0)),
        ],
        out_specs=[pl.BlockSpec((BEP, 128), lambda i: (i, 0))] * 2,
        out_shape=[jax.ShapeDtypeStruct((E // 2, 128), F32)] * 2,
    )(eae, eao, w2T, weT, cb)


def _s0_body(dst_hbm, ea_hbm, acc_hbm,
             di0, di1, drs0, drs1, eab0, eab1, wbuf0, wbuf1, zb, acc_sh,
             semi0, semi1, sems0, sems1):
    cid = lax.axis_index("c")
    sid = lax.axis_index("s")
    wid = cid * NS + sid
    lanes = lax.iota(I32, 16)
    mask0 = jnp.where(lanes == 0, 1.0, 0.0).astype(F32)

    di = (di0, di1)
    drs = (drs0, drs1)
    eab = (eab0, eab1)
    wbuf = (wbuf0, wbuf1)
    semi = (semi0, semi1)
    sems = (sems0, sems1)

    for b in range(2):
        _zero_vmem(wbuf[b], 128)

        def fill(i, _, _b=b):
            wbuf[_b][i, pl.ds(16, 16)] = mask0
            return 0
        lax.fori_loop(0, CB1, fill, 0)
    _zero_vmem(zb, 128)
    z16i = jnp.zeros((16,), I32)
    for b in range(2):
        for g in range(CB1 // 16 + 1):
            drs[b][pl.ds(min(g * 16, CB1 - 16), 16)] = z16i
    _zero_shared(zb, acc_sh, sid)
    plsc.subcore_barrier()

    nch = _EPW // CB1
    ebase = wid * _EPW

    def fire_idx(c, b):
        bs = _m8(ebase + c * CB1)
        pltpu.async_copy(dst_hbm.at[pl.ds(bs, CB1)], di[b], semi[b])
        pltpu.async_copy(ea_hbm.at[pl.ds(_m8(bs * ED), CB1 * ED)],
                         eab[b], semi[b])

    def wait_idx(b):
        pltpu.make_async_copy(dst_hbm.at[pl.ds(0, CB1)], di[b],
                              semi[b]).wait()
        pltpu.make_async_copy(ea_hbm.at[pl.ds(0, CB1 * ED)], eab[b],
                              semi[b]).wait()

    def wait_scatter(b):
        pltpu.make_async_copy(acc_hbm.at[pl.ds(0, CB1)], wbuf[b],
                              sems[b]).wait()

    for b in range(2):
        pltpu.async_copy(wbuf[b], acc_sh.at[drs[b]], sems[b], add=True)
        fire_idx(b, b)

    def body(k2, _):
        for b in range(2):
            c = 2 * k2 + b
            wait_scatter(b)
            wait_idx(b)
            for g in range(CB1 // 16 + 1):
                sl = pl.ds(min(g * 16, CB1 - 16), 16)
                drs[b][sl] = di[b][sl]
            wb = wbuf[b]
            ea = eab[b]

            def edge(j, _):
                wb[j, pl.ds(0, 16)] = ea[pl.ds(j * ED, 16)]
                return 0
            lax.fori_loop(0, CB1, edge, 0)
            pltpu.async_copy(wb, acc_sh.at[drs[b]], sems[b], add=True)
            cn = c + 2
            cc = jnp.where(cn < nch, cn, 0)
            fire_idx(cc, b)
        return 0
    lax.fori_loop(0, nch // 2, body, 0)
    for b in range(2):
        wait_scatter(b)
        wait_idx(b)
    plsc.subcore_barrier()

    _copy_out_shared(acc_sh, acc_hbm, cid, sid)


def _s0(dst, eaf):
    return pl.kernel(
        _s0_body,
        out_type=jax.ShapeDtypeStruct((NC * N, 128), F32),
        mesh=plsc.VectorSubcoreMesh(**_MESH),
        scratch_types=[
            pltpu.VMEM((CB1,), I32),
            pltpu.VMEM((CB1,), I32),
            pltpu.VMEM((CB1,), I32),
            pltpu.VMEM((CB1,), I32),
            pltpu.VMEM((CB1 * ED,), F32),
            pltpu.VMEM((CB1 * ED,), F32),
            pltpu.VMEM((CB1, 128), F32),
            pltpu.VMEM((CB1, 128), F32),
            pltpu.VMEM((_SEGC, 128), F32),
            pltpu.VMEM_SHARED((N, 128), F32),
            pltpu.SemaphoreType.DMA,
            pltpu.SemaphoreType.DMA,
            pltpu.SemaphoreType.DMA,
            pltpu.SemaphoreType.DMA,
        ],
    )(dst, eaf)


def _s1h_body(t, soff_hbm, doff_hbm, dstr_hbm, tbl_hbm, ep_hbm, att_hbm,
              acc_hbm,
              si0, si1, di0, di1, dr0, dr1, drs0, drs1,
              srow0, srow1, drow0, drow1, epb, wbuf0, wbuf1,
              attb, zb, acc_sh,
              semi0, semi1, semg0, semg1, sems0, sems1, semp):
    cid = lax.axis_index("c")
    sid = lax.axis_index("s")
    lanes = lax.iota(I32, 16)
    mask0 = jnp.where(lanes == 0, 1.0, 0.0).astype(F32)
    head = 2 * t + cid

    si = (si0, si1)
    di = (di0, di1)
    dr = (dr0, dr1)
    drs = (drs0, drs1)
    srow = (srow0, srow1)
    drow = (drow0, drow1)
    wbuf = (wbuf0, wbuf1)
    semi = (semi0, semi1)
    semg = (semg0, semg1)
    sems = (sems0, sems1)

    pltpu.sync_copy(att_hbm, attb)
    att_vecs = [attb[pl.ds(head * HID + q * 16, 16)] for q in range(4)]

    for b in range(2):
        _zero_vmem(wbuf[b], 128)
    _zero_vmem(zb, 128)
    z16i = jnp.zeros((16,), I32)
    for b in range(2):
        for g in range(CB1 // 16 + 1):
            drs[b][pl.ds(min(g * 16, CB1 - 16), 16)] = z16i
    _zero_shared(zb, acc_sh, sid)
    plsc.subcore_barrier()

    nch = _EPT // CB1
    ebase = head * E + sid * _EPT
    rbase = sid * _EPT

    def fire_idx(c, b):
        bs = _m8(ebase + c * CB1)
        br = _m8(rbase + c * CB1)
        pltpu.async_copy(soff_hbm.at[pl.ds(bs, CB1)], si[b], semi[b])
        pltpu.async_copy(doff_hbm.at[pl.ds(bs, CB1)], di[b], semi[b])
        pltpu.async_copy(dstr_hbm.at[pl.ds(br, CB1)], dr[b], semi[b])

    def wait_idx(b):
        pltpu.make_async_copy(soff_hbm.at[pl.ds(0, CB1)], si[b], semi[b]).wait()
        pltpu.make_async_copy(doff_hbm.at[pl.ds(0, CB1)], di[b], semi[b]).wait()
        pltpu.make_async_copy(dstr_hbm.at[pl.ds(0, CB1)], dr[b], semi[b]).wait()

    def fire_gather(c, b):
        pltpu.async_copy(tbl_hbm.at[si[b]], srow[b], semg[b])
        pltpu.async_copy(tbl_hbm.at[di[b]], drow[b], semg[b])

    def wait_gather(b):
        pltpu.make_async_copy(tbl_hbm.at[si[b]], srow[b], semg[b]).wait()
        pltpu.make_async_copy(tbl_hbm.at[di[b]], drow[b], semg[b]).wait()

    def wait_scatter(b):
        pltpu.make_async_copy(acc_hbm.at[pl.ds(0, CB1)], wbuf[b],
                              sems[b]).wait()

    for b in range(2):
        pltpu.async_copy(wbuf[b], acc_sh.at[drs[b]], sems[b], add=True)
        fire_idx(b, b)
    for b in range(2):
        wait_idx(b)
        fire_gather(b, b)

    def body(k2, _):
        bp = _m8((ebase + 2 * k2 * CB1) // 2)
        cpe = pltpu.async_copy(ep_hbm.at[pl.ds(bp, CB1)], epb, semp)
        for b in range(2):
            c = 2 * k2 + b
            wait_scatter(b)
            wait_gather(b)
            if b == 0:
                cpe.wait()
            for g in range(CB1 // 16 + 1):
                sl = pl.ds(min(g * 16, CB1 - 16), 16)
                drs[b][sl] = dr[b][sl]
            cn = c + 2
            cc = jnp.where(cn < nch, cn, 0)
            fire_idx(cc, b)
            sr = srow[b]
            drr = drow[b]
            wb = wbuf[b]
            prow = b * (CB1 // 2)

            def pair(p, _):
                for u in range(2):
                    j = 2 * p + u
                    acc = _zeros16()
                    for q in range(4):
                        s = (sr[j, pl.ds(q * 16, 16)]
                             + drr[j, pl.ds(64 + q * 16, 16)]
                             + epb[prow + p, pl.ds(u * 64 + q * 16, 16)])
                        m = jnp.maximum(s, 0.2 * s)
                        acc = acc + m * att_vecs[q]
                    ex = jnp.exp(_allsum(acc))
                    for q in range(4):
                        sl = pl.ds(q * 16, 16)
                        wb[j, sl] = sr[j, sl] * ex
                    wb[j, pl.ds(64, 16)] = ex * mask0
                return 0
            lax.fori_loop(0, CB1 // 2, pair, 0)
            pltpu.async_copy(wb, acc_sh.at[drs[b]], sems[b], add=True)
            wait_idx(b)
            fire_gather(cc, b)
        return 0
    lax.fori_loop(0, nch // 2, body, 0)
    for b in range(2):
        wait_scatter(b)
        wait_gather(b)
    plsc.subcore_barrier()

    _copy_out_shared(acc_sh, acc_hbm, cid, sid)


def _s1h(t, soff, doff, dstr, tbl, ep, attf):
    return pl.kernel(
        functools.partial(_s1h_body, t),
        out_type=jax.ShapeDtypeStruct((NC * N, 128), F32),
        mesh=plsc.VectorSubcoreMesh(**_MESH),
        scratch_types=[
            pltpu.VMEM((CB1,), I32),
            pltpu.VMEM((CB1,), I32),
            pltpu.VMEM((CB1,), I32),
            pltpu.VMEM((CB1,), I32),
            pltpu.VMEM((CB1,), I32),
            pltpu.VMEM((CB1,), I32),
            pltpu.VMEM((CB1,), I32),
            pltpu.VMEM((CB1,), I32),
            pltpu.VMEM((CB1, 128), F32),
            pltpu.VMEM((CB1, 128), F32),
            pltpu.VMEM((CB1, 128), F32),
            pltpu.VMEM((CB1, 128), F32),
            pltpu.VMEM((CB1, 128), F32),
            pltpu.VMEM((CB1, 128), F32),
            pltpu.VMEM((CB1, 128), F32),
            pltpu.VMEM((256,), F32),
            pltpu.VMEM((_SEGC, 128), F32),
            pltpu.VMEM_SHARED((N, 128), F32),
            pltpu.SemaphoreType.DMA,
            pltpu.SemaphoreType.DMA,
            pltpu.SemaphoreType.DMA,
            pltpu.SemaphoreType.DMA,
            pltpu.SemaphoreType.DMA,
            pltpu.SemaphoreType.DMA,
            pltpu.SemaphoreType.DMA,
        ],
    )(soff, doff, dstr, tbl, ep, attf)


def _s2_body(src_hbm, dst_hbm, tbl_hbm, ep_hbm, att_hbm,
             acc_hbm,
             si0, si1, di0, di1, drs0, drs1,
             srow0, srow1, drow0, drow1, epb, wbuf0, wbuf1,
             attb, zb, acc_sh,
             semi0, semi1, semg0, semg1, sems0, sems1, semp):
    cid = lax.axis_index("c")
    sid = lax.axis_index("s")
    wid = cid * NS + sid
    lanes = lax.iota(I32, 16)
    mask0 = jnp.where(lanes == 0, 1.0, 0.0).astype(F32)

    si = (si0, si1)
    di = (di0, di1)
    drs = (drs0, drs1)
    srow = (srow0, srow1)
    drow = (drow0, drow1)
    wbuf = (wbuf0, wbuf1)
    semi = (semi0, semi1)
    semg = (semg0, semg1)
    sems = (sems0, sems1)

    pltpu.sync_copy(att_hbm, attb)
    att_vecs = [attb[pl.ds(q * 16, 16)] for q in range(4)]

    for b in range(2):
        _zero_vmem(wbuf[b], 128)
    _zero_vmem(zb, 128)
    z16i = jnp.zeros((16,), I32)
    for b in range(2):
        for g in range(CB1 // 16 + 1):
            drs[b][pl.ds(min(g * 16, CB1 - 16), 16)] = z16i
    _zero_shared(zb, acc_sh, sid)
    plsc.subcore_barrier()

    nch = _EPW // CB1
    ebase = wid * _EPW

    def fire_idx(c, b):
        bs = _m8(ebase + c * CB1)
        pltpu.async_copy(src_hbm.at[pl.ds(bs, CB1)], si[b], semi[b])
        pltpu.async_copy(dst_hbm.at[pl.ds(bs, CB1)], di[b], semi[b])

    def wait_idx(b):
        pltpu.make_async_copy(src_hbm.at[pl.ds(0, CB1)], si[b], semi[b]).wait()
        pltpu.make_async_copy(dst_hbm.at[pl.ds(0, CB1)], di[b], semi[b]).wait()

    def fire_gather(c, b):
        pltpu.async_copy(tbl_hbm.at[si[b]], srow[b], semg[b])
        pltpu.async_copy(tbl_hbm.at[di[b]], drow[b], semg[b])

    def wait_gather(b):
        pltpu.make_async_copy(tbl_hbm.at[si[b]], srow[b], semg[b]).wait()
        pltpu.make_async_copy(tbl_hbm.at[di[b]], drow[b], semg[b]).wait()

    def wait_scatter(b):
        pltpu.make_async_copy(acc_hbm.at[pl.ds(0, CB1)], wbuf[b],
                              sems[b]).wait()

    for b in range(2):
        pltpu.async_copy(wbuf[b], acc_sh.at[drs[b]], sems[b], add=True)
        fire_idx(b, b)
    for b in range(2):
        wait_idx(b)
        fire_gather(b, b)

    def body(k2, _):
        bp = _m8((ebase + 2 * k2 * CB1) // 2)
        cpe = pltpu.async_copy(ep_hbm.at[pl.ds(bp, CB1)], epb, semp)
        for b in range(2):
            c = 2 * k2 + b
            wait_scatter(b)
            wait_gather(b)
            if b == 0:
                cpe.wait()
            for g in range(CB1 // 16 + 1):
                sl = pl.ds(min(g * 16, CB1 - 16), 16)
                drs[b][sl] = di[b][sl]
            cn = c + 2
            cc = jnp.where(cn < nch, cn, 0)
            fire_idx(cc, b)
            sr = srow[b]
            drr = drow[b]
            wb = wbuf[b]
            prow = b * (CB1 // 2)

            def pair(p, _):
                for u in range(2):
                    j = 2 * p + u
                    acc = _zeros16()
                    for q in range(4):
                        s = (sr[j, pl.ds(q * 16, 16)]
                             + drr[j, pl.ds(64 + q * 16, 16)]
                             + epb[prow + p, pl.ds(u * 64 + q * 16, 16)])
                        m = jnp.maximum(s, 0.2 * s)
                        acc = acc + m * att_vecs[q]
                    ex = jnp.exp(_allsum(acc))
                    for q in range(4):
                        sl = pl.ds(q * 16, 16)
                        wb[j, sl] = sr[j, sl] * ex
                    wb[j, pl.ds(64, 16)] = ex * mask0
                return 0
            lax.fori_loop(0, CB1 // 2, pair, 0)
            pltpu.async_copy(wb, acc_sh.at[drs[b]], sems[b], add=True)
            wait_idx(b)
            fire_gather(cc, b)
        return 0
    lax.fori_loop(0, nch // 2, body, 0)
    for b in range(2):
        wait_scatter(b)
        wait_gather(b)
    plsc.subcore_barrier()

    _copy_out_shared(acc_sh, acc_hbm, cid, sid)


def _s2(src, dst, tbl, ep, attf):
    return pl.kernel(
        _s2_body,
        out_type=jax.ShapeDtypeStruct((NC * N, 128), F32),
        mesh=plsc.VectorSubcoreMesh(**_MESH),
        scratch_types=[
            pltpu.VMEM((CB1,), I32),
            pltpu.VMEM((CB1,), I32),
            pltpu.VMEM((CB1,), I32),
            pltpu.VMEM((CB1,), I32),
            pltpu.VMEM((CB1,), I32),
            pltpu.VMEM((CB1,), I32),
            pltpu.VMEM((CB1, 128), F32),
            pltpu.VMEM((CB1, 128), F32),
            pltpu.VMEM((CB1, 128), F32),
            pltpu.VMEM((CB1, 128), F32),
            pltpu.VMEM((CB1, 128), F32),
            pltpu.VMEM((CB1, 128), F32),
            pltpu.VMEM((CB1, 128), F32),
            pltpu.VMEM((64,), F32),
            pltpu.VMEM((_SEGC, 128), F32),
            pltpu.VMEM_SHARED((N, 128), F32),
            pltpu.SemaphoreType.DMA,
            pltpu.SemaphoreType.DMA,
            pltpu.SemaphoreType.DMA,
            pltpu.SemaphoreType.DMA,
            pltpu.SemaphoreType.DMA,
            pltpu.SemaphoreType.DMA,
            pltpu.SemaphoreType.DMA,
        ],
    )(src, dst, tbl, ep, attf)


def _s3_body(src_hbm, dst_hbm, g_hbm, eec_hbm, cw_hbm,
             lg_hbm,
             si0, si1, di0, di1, srow0, srow1, drow0, drow1, ecb0, ecb1,
             wb_, lbuf, sem0, semi0, semi1, semg0, semg1):
    cid = lax.axis_index("c")
    sid = lax.axis_index("s")
    wid = cid * NS + sid
    lanes = lax.iota(I32, 16)

    si = (si0, si1)
    di = (di0, di1)
    srow = (srow0, srow1)
    drow = (drow0, drow1)
    ecb = (ecb0, ecb1)
    semi = (semi0, semi1)
    semg = (semg0, semg1)

    pltpu.sync_copy(cw_hbm, wb_)
    w_vecs = [wb_[pl.ds(q * 16, 16)] for q in range(4)]
    cb_vec = wb_[pl.ds(64, 16)]

    nch = _EPW // CB
    ebase = wid * _EPW

    def fire_idx(c, b):
        bs = _m8(ebase + c * CB)
        pltpu.async_copy(src_hbm.at[pl.ds(bs, CB)], si[b], semi[b])
        pltpu.async_copy(dst_hbm.at[pl.ds(bs, CB)], di[b], semi[b])
        pltpu.async_copy(eec_hbm.at[pl.ds(_m8(bs // 2), CB // 2)],
                         ecb[b], semi[b])

    def wait_idx(b):
        pltpu.make_async_copy(src_hbm.at[pl.ds(0, CB)], si[b], semi[b]).wait()
        pltpu.make_async_copy(dst_hbm.at[pl.ds(0, CB)], di[b], semi[b]).wait()
        pltpu.make_async_copy(eec_hbm.at[pl.ds(0, CB // 2)], ecb[b],
                              semi[b]).wait()

    def fire_gather(b):
        pltpu.async_copy(g_hbm.at[si[b]], srow[b], semg[b])
        pltpu.async_copy(g_hbm.at[di[b]], drow[b], semg[b])

    def wait_gather(b):
        pltpu.make_async_copy(g_hbm.at[pl.ds(0, CB)], srow[b], semg[b]).wait()
        pltpu.make_async_copy(g_hbm.at[pl.ds(0, CB)], drow[b], semg[b]).wait()

    def compute_write(c, b):
        sr = srow[b]
        drr = drow[b]
        ec = ecb[b]

        def group(g, _):
            lvec = cb_vec
            for jj in range(16):
                j = g * 16 + jj
                acc = _zeros16()
                for q in range(4):
                    z = (sr[j, pl.ds(q * 16, 16)]
                         + drr[j, pl.ds(64 + q * 16, 16)]
                         + ec[(g * 16 + jj) // 2,
                              pl.ds((jj % 2) * 64 + q * 16, 16)])
                    z = jnp.maximum(z, 0.0)
                    acc = acc + z * w_vecs[q]
                a = _allsum(acc)
                mj = jnp.where(lanes == jj, 1.0, 0.0).astype(F32)
                lvec = lvec + a * mj
            lbuf[pl.ds(g * 16, 16)] = lvec
            return 0
        lax.fori_loop(0, CB // 16, group, 0)
        pltpu.sync_copy(lbuf, lg_hbm.at[pl.ds(_m8(ebase + c * CB), CB)])

    fire_idx(0, 0)
    fire_idx(1, 1)
    wait_idx(0)
    fire_gather(0)
    wait_idx(1)
    fire_gather(1)
    wait_gather(0)
    compute_write(0, 0)
    fire_idx(2, 0)

    def body(k2, _):
        ca = 1 + 2 * k2
        wait_gather(1)
        wait_idx(0)
        fire_gather(0)
        compute_write(ca, 1)
        cn = ca + 2
        fire_idx(jnp.where(cn < nch, cn, 0), 1)
        wait_gather(0)
        wait_idx(1)
        fire_gather(1)
        compute_write(ca + 1, 0)
        cn2 = ca + 3
        fire_idx(jnp.where(cn2 < nch, cn2, 0), 0)
        return 0
    lax.fori_loop(0, (nch - 1) // 2, body, 0)
    wait_gather(1)
    wait_idx(0)


def _s3(src, dst, g, eec, cwm):
    return pl.kernel(
        _s3_body,
        out_type=jax.ShapeDtypeStruct((E,), F32),
        mesh=plsc.VectorSubcoreMesh(**_MESH),
        scratch_types=[
            pltpu.VMEM((CB,), I32),
            pltpu.VMEM((CB,), I32),
            pltpu.VMEM((CB,), I32),
            pltpu.VMEM((CB,), I32),
            pltpu.VMEM((CB, 128), F32),
            pltpu.VMEM((CB, 128), F32),
            pltpu.VMEM((CB, 128), F32),
            pltpu.VMEM((CB, 128), F32),
            pltpu.VMEM((CB // 2, 128), F32),
            pltpu.VMEM((CB // 2, 128), F32),
            pltpu.VMEM((80,), F32),
            pltpu.VMEM((CB,), F32),
            pltpu.SemaphoreType.DMA,
            pltpu.SemaphoreType.DMA,
            pltpu.SemaphoreType.DMA,
            pltpu.SemaphoreType.DMA,
            pltpu.SemaphoreType.DMA,
        ],
    )(src, dst, g, eec, cwm)


def _ep1_body(acca_ref, accb_ref, s0_ref, xlr_ref,
              ewT_ref, s16_ref, s64_ref, attm_ref, bias_ref,
              l2lwT_ref, l2lb_ref, l2rwT_ref, l2rb_ref,
              c2_ref, la_ref):
    s0s = s0_ref[0] + s0_ref[1]
    ea = s0s[:, :ED]
    cntv = jnp.dot(s0s, s16_ref[...], preferred_element_type=F32)
    la = ea / jnp.maximum(cntv, 1.0)
    la_ref[...] = la
    lp = jnp.dot(la, ewT_ref[...], preferred_element_type=F32)

    accs = [acca_ref[0], acca_ref[1], accb_ref[0], accb_ref[1]]
    xls = []
    mls = []
    for h in range(4):
        xlh = xlr_ref[h][:, :HID]
        xrh = xlr_ref[h][:, HID:]
        mm = xlh + xrh + lp[:, h * HID:(h + 1) * HID]
        xls.append(xlh)
        mls.append(jnp.maximum(mm, 0.2 * mm))
    al = sum(jnp.dot(mls[h], attm_ref[h * HID:(h + 1) * HID, :],
                     preferred_element_type=F32) for h in range(4))
    exl = jnp.exp(al)
    xl2v = l2lb_ref[...]
    xr2v = l2rb_ref[...]
    for h in range(4):
        exh = exl[:, h:h + 1]
        denh = jnp.dot(accs[h], s64_ref[...], preferred_element_type=F32)
        num = accs[h][:, :HID] + exh * xls[h]
        hv = num / (denh + exh + 1e-16) + bias_ref[:, h * HID:(h + 1) * HID]
        hv = jnp.where(hv > 0, hv, jnp.exp(jnp.minimum(hv, 0.0)) - 1.0)
        xl2v = xl2v + jnp.dot(hv, l2lwT_ref[h * HID:(h + 1) * HID, :],
                              preferred_element_type=F32)
        xr2v = xr2v + jnp.dot(hv, l2rwT_ref[h * HID:(h + 1) * HID, :],
                              preferred_element_type=F32)
    c2_ref[...] = jnp.concatenate([xl2v, xr2v], axis=1)


def _ep1(acca, accb, s0acc, xlr, ewT, s16, s64, attm, bias,
         l2lwT, l2lb, l2rwT, l2rb):
    def full(shape):
        return pl.BlockSpec(shape, lambda i, _n=len(shape): (0,) * _n)
    return pl.pallas_call(
        _ep1_body,
        grid=(N // BN,),
        in_specs=[
            pl.BlockSpec((NC, BN, 128), lambda i: (0, i, 0)),
            pl.BlockSpec((NC, BN, 128), lambda i: (0, i, 0)),
            pl.BlockSpec((NC, BN, 128), lambda i: (0, i, 0)),
            pl.BlockSpec((4, BN, 128), lambda i: (0, i, 0)),
            full((ED, 256)),
            full((128, 1)),
            full((128, 1)),
            full((256, 4)),
            full((1, 256)),
            full((256, HID)),
            full((1, HID)),
            full((256, HID)),
            full((1, HID)),
        ],
        out_specs=[pl.BlockSpec((BN, 128), lambda i: (i, 0)),
                   pl.BlockSpec((BN, ED), lambda i: (i, 0))],
        out_shape=[jax.ShapeDtypeStruct((N, 128), F32),
                   jax.ShapeDtypeStruct((N, ED), F32)],
    )(acca, accb, s0acc, xlr, ewT, s16, s64, attm, bias,
      l2lwT, l2lb, l2rwT, l2rb)


def _ep2_body(acc_ref, c2_ref, la_ref,
              ew2T_ref, att2_ref, s64_ref, bias_ref, wsT_ref, wdT_ref,
              g_ref):
    accs = acc_ref[0] + acc_ref[1]
    den = jnp.dot(accs, s64_ref[...], preferred_element_type=F32)
    lp2 = jnp.dot(la_ref[...], ew2T_ref[...], preferred_element_type=F32)
    xl2 = c2_ref[:, :HID]
    xr2 = c2_ref[:, HID:]
    mm = xl2 + xr2 + lp2
    ml = jnp.maximum(mm, 0.2 * mm)
    al = jnp.dot(ml, att2_ref[...], preferred_element_type=F32)
    ex = jnp.exp(al)
    h2 = (accs[:, :HID] + ex * xl2) / (den + ex + 1e-16) + bias_ref[...]
    h2 = jnp.where(h2 > 0, h2, jnp.exp(jnp.minimum(h2, 0.0)) - 1.0)
    g_ref[...] = jnp.concatenate(
        [jnp.dot(h2, wsT_ref[...], preferred_element_type=F32),
         jnp.dot(h2, wdT_ref[...], preferred_element_type=F32)], axis=1)


def _ep2(acc, c2, la, ew2T, att2, s64, bias, wsT, wdT):
    def full(shape):
        return pl.BlockSpec(shape, lambda i, _n=len(shape): (0,) * _n)
    return pl.pallas_call(
        _ep2_body,
        grid=(N // BN,),
        in_specs=[
            pl.BlockSpec((NC, BN, 128), lambda i: (0, i, 0)),
            pl.BlockSpec((BN, 128), lambda i: (i, 0)),
            pl.BlockSpec((BN, ED), lambda i: (i, 0)),
            full((ED, HID)),
            full((HID, 1)),
            full((128, 1)),
            full((1, HID)),
            full((HID, HID)),
            full((HID, HID)),
        ],
        out_specs=pl.BlockSpec((BN, 128), lambda i: (i, 0)),
        out_shape=jax.ShapeDtypeStruct((N, 128), F32),
    )(acc, c2, la, ew2T, att2, s64, bias, wsT, wdT)


_S16 = np.zeros((128, 1), np.float32)
_S16[16, 0] = 1.0
_S64 = np.zeros((128, 1), np.float32)
_S64[64, 0] = 1.0


def kernel(x, edge_index, edge_attr, l1_lw, l1_lb, l1_rw, l1_rb, l1_ew,
           l1_att, l1_bias, l2_lw, l2_lb, l2_rw, l2_rb, l2_ew, l2_att,
           l2_bias, c1_w, c1_b, c2_w, c2_b):
    src = edge_index[0]
    dst = edge_index[1]
    eae = edge_attr[0::2]
    eao = edge_attr[1::2]
    eaf = edge_attr.reshape(-1)

    def hsplit(w):
        return w.reshape(w.shape[0], 4, HID).transpose(1, 0, 2)

    tbl1 = _t1(x, hsplit(l1_lw.T), l1_lb.reshape(4, 1, HID),
               hsplit(l1_rw.T), l1_rb.reshape(4, 1, HID))
    ep1 = _t2a(eae, eao, hsplit(l1_ew.T))
    ep2, eec = _t2b(eae, eao, l2_ew.T, c1_w[:, 128:].T, c1_b[None])

    s0acc = _s0(dst, eaf)

    tblf = tbl1.reshape(4 * N, 128)
    epf = ep1.reshape(4 * (E // 2), 128)
    attm1 = l1_att.reshape(-1)
    hoff = (jnp.arange(4, dtype=I32) * N)[:, None]
    srcoff = (src[None, :] + hoff).reshape(-1)
    dstoff = (dst[None, :] + hoff).reshape(-1)
    acca = _s1h(0, srcoff, dstoff, dst, tblf, epf, attm1)
    accb = _s1h(1, srcoff, dstoff, dst, tblf, epf, attm1)

    attmask = jnp.zeros((256, 4), F32).at[
        jnp.arange(256), jnp.arange(256) // 64].set(l1_att.reshape(-1))
    c2tbl, la = _ep1(
        acca.reshape(NC, N, 128), accb.reshape(NC, N, 128),
        s0acc.reshape(NC, N, 128), tbl1,
        l1_ew.T, jnp.asarray(_S16), jnp.asarray(_S64),
        attmask, l1_bias[None],
        l2_lw.T, l2_lb[None], l2_rw.T, l2_rb[None])

    acc2 = _s2(src, dst, c2tbl, ep2, l2_att.reshape(-1))

    gtbl = _ep2(acc2.reshape(NC, N, 128), c2tbl, la,
                l2_ew.T, l2_att.reshape(HID, 1), jnp.asarray(_S64),
                l2_bias[None], c1_w[:, :64].T, c1_w[:, 64:128].T)

    cwm = jnp.concatenate([c2_w.reshape(-1),
                           jnp.broadcast_to(c2_b, (16,))])
    logits = _s3(src, dst, gtbl, eec, cwm)
    return logits


# speedup vs baseline: 13.5348x; 1.0004x over previous
import functools

import jax
import jax.numpy as jnp
import numpy as np
from jax import lax
from jax.experimental import pallas as pl
from jax.experimental.pallas import tpu as pltpu
from jax.experimental.pallas import tpu_sc as plsc

N = 10000
E = 320000
ND = 128
ED = 16
HID = 64
NC = 2
NS = 16
CB = 80
CB1 = 40
BN = 1000
BEP = 2000

F32 = jnp.float32
I32 = jnp.int32

_MESH = dict(core_axis_name="c", subcore_axis_name="s")

_EPW = E // (NC * NS)
_EPT = E // NS
_SEG = 624
_SEGC = 104
_TAIL = N - NS * _SEG

_DNUMS = lax.GatherDimensionNumbers(
    offset_dims=(), collapsed_slice_dims=(0,), start_index_map=(0,))


def _allsum(v):
    lanes = lax.iota(I32, 16)
    for k in range(4):
        p = lax.bitwise_xor(lanes, 1 << k)
        v = v + lax.gather(v, p[:, None], _DNUMS, slice_sizes=(1,),
                           mode=lax.GatherScatterMode.PROMISE_IN_BOUNDS)
    return v


def _zeros16():
    return jnp.zeros((16,), F32)


def _m8(v):
    return pl.multiple_of(v, 8)


def _zero_vmem(buf, w):
    z16 = _zeros16()

    def zr(i, _):
        for q in range(w // 16):
            buf[i, pl.ds(q * 16, 16)] = z16
        return 0
    lax.fori_loop(0, buf.shape[0], zr, 0)


def _zero_shared(zb, sh, sid):
    for t in range(_SEG // _SEGC):
        pltpu.sync_copy(zb, sh.at[pl.ds(_m8(sid * _SEG + t * _SEGC), _SEGC)])

    @pl.when(sid == NS - 1)
    def _():
        pltpu.sync_copy(zb.at[pl.ds(0, _TAIL)], sh.at[pl.ds(NS * _SEG, _TAIL)])


def _copy_out_shared(sh, hbm, cid, sid):
    pltpu.sync_copy(sh.at[pl.ds(_m8(sid * _SEG), _SEG)],
                    hbm.at[pl.ds(_m8(cid * N + sid * _SEG), _SEG)])

    @pl.when(sid == NS - 1)
    def _():
        pltpu.sync_copy(sh.at[pl.ds(NS * _SEG, _TAIL)],
                        hbm.at[pl.ds(_m8(cid * N + NS * _SEG), _TAIL)])


def _t1_body(x_ref, lwT_ref, lb_ref, rwT_ref, rb_ref, c_ref):
    xv = x_ref[...]
    xl = jnp.dot(xv, lwT_ref[0], preferred_element_type=F32) + lb_ref[0]
    xr = jnp.dot(xv, rwT_ref[0], preferred_element_type=F32) + rb_ref[0]
    c_ref[...] = jnp.concatenate([xl, xr], axis=1)[None]


def _t1(x, lwT, lb, rwT, rb):
    return pl.pallas_call(
        _t1_body,
        grid=(4, N // BN),
        in_specs=[
            pl.BlockSpec((BN, ND), lambda c, i: (i, 0)),
            pl.BlockSpec((1, ND, HID), lambda c, i: (c, 0, 0)),
            pl.BlockSpec((1, 1, HID), lambda c, i: (c, 0, 0)),
            pl.BlockSpec((1, ND, HID), lambda c, i: (c, 0, 0)),
            pl.BlockSpec((1, 1, HID), lambda c, i: (c, 0, 0)),
        ],
        out_specs=pl.BlockSpec((1, BN, 128), lambda c, i: (c, i, 0)),
        out_shape=jax.ShapeDtypeStruct((4, N, 128), F32),
    )(x, lwT, lb, rwT, rb)


def _t2a_body(eae_ref, eao_ref, ewT_ref, ep_ref):
    pe = jnp.dot(eae_ref[...], ewT_ref[0], preferred_element_type=F32)
    po = jnp.dot(eao_ref[...], ewT_ref[0], preferred_element_type=F32)
    ep_ref[...] = jnp.concatenate([pe, po], axis=1)[None]


def _t2a(eae, eao, ewT):
    return pl.pallas_call(
        _t2a_body,
        grid=(4, (E // 2) // BEP),
        in_specs=[
            pl.BlockSpec((BEP, ED), lambda c, i: (i, 0)),
            pl.BlockSpec((BEP, ED), lambda c, i: (i, 0)),
            pl.BlockSpec((1, ED, HID), lambda c, i: (c, 0, 0)),
        ],
        out_specs=pl.BlockSpec((1, BEP, 128), lambda c, i: (c, i, 0)),
        out_shape=jax.ShapeDtypeStruct((4, E // 2, 128), F32),
    )(eae, eao, ewT)


def _t2b_body(eae_ref, eao_ref, w2T_ref, weT_ref, cb_ref, ep2_ref, eec_ref):
    eav = eae_ref[...]
    eov = eao_ref[...]
    ep2_ref[...] = jnp.concatenate(
        [jnp.dot(eav, w2T_ref[...], preferred_element_type=F32),
         jnp.dot(eov, w2T_ref[...], preferred_element_type=F32)], axis=1)
    eec_ref[...] = jnp.concatenate(
        [jnp.dot(eav, weT_ref[...], preferred_element_type=F32) + cb_ref[...],
         jnp.dot(eov, weT_ref[...], preferred_element_type=F32) + cb_ref[...]],
        axis=1)


def _t2b(eae, eao, w2T, weT, cb):
    return pl.pallas_call(
        _t2b_body,
        grid=((E // 2) // BEP,),
        in_specs=[
            pl.BlockSpec((BEP, ED), lambda i: (i, 0)),
            pl.BlockSpec((BEP, ED), lambda i: (i, 0)),
            pl.BlockSpec((ED, HID), lambda i: (0, 0)),
            pl.BlockSpec((ED, HID), lambda i: (0, 0)),
            pl.BlockSpec((1, HID), lambda i: (0, 0)),
        ],
        out_specs=[pl.BlockSpec((BEP, 128), lambda i: (i, 0))] * 2,
        out_shape=[jax.ShapeDtypeStruct((E // 2, 128), F32)] * 2,
    )(eae, eao, w2T, weT, cb)


def _s0_body(dst_hbm, ea_hbm, acc_hbm,
             di0, di1, drs0, drs1, eab0, eab1, wbuf0, wbuf1, zb, acc_sh,
             semi0, semi1, sems0, sems1):
    cid = lax.axis_index("c")
    sid = lax.axis_index("s")
    wid = cid * NS + sid
    lanes = lax.iota(I32, 16)
    mask0 = jnp.where(lanes == 0, 1.0, 0.0).astype(F32)

    di = (di0, di1)
    drs = (drs0, drs1)
    eab = (eab0, eab1)
    wbuf = (wbuf0, wbuf1)
    semi = (semi0, semi1)
    sems = (sems0, sems1)

    for b in range(2):
        _zero_vmem(wbuf[b], 128)

        def fill(i, _, _b=b):
            wbuf[_b][i, pl.ds(16, 16)] = mask0
            return 0
        lax.fori_loop(0, CB1, fill, 0)
    _zero_vmem(zb, 128)
    z16i = jnp.zeros((16,), I32)
    for b in range(2):
        for g in range(CB1 // 16 + 1):
            drs[b][pl.ds(min(g * 16, CB1 - 16), 16)] = z16i
    _zero_shared(zb, acc_sh, sid)
    plsc.subcore_barrier()

    nch = _EPW // CB1
    ebase = wid * _EPW

    def fire_idx(c, b):
        bs = _m8(ebase + c * CB1)
        pltpu.async_copy(dst_hbm.at[pl.ds(bs, CB1)], di[b], semi[b])
        pltpu.async_copy(ea_hbm.at[pl.ds(_m8(bs * ED), CB1 * ED)],
                         eab[b], semi[b])

    def wait_idx(b):
        pltpu.make_async_copy(dst_hbm.at[pl.ds(0, CB1)], di[b],
                              semi[b]).wait()
        pltpu.make_async_copy(ea_hbm.at[pl.ds(0, CB1 * ED)], eab[b],
                              semi[b]).wait()

    def wait_scatter(b):
        pltpu.make_async_copy(acc_hbm.at[pl.ds(0, CB1)], wbuf[b],
                              sems[b]).wait()

    for b in range(2):
        pltpu.async_copy(wbuf[b], acc_sh.at[drs[b]], sems[b], add=True)
        fire_idx(b, b)

    def body(k2, _):
        for b in range(2):
            c = 2 * k2 + b
            wait_scatter(b)
            wait_idx(b)
            for g in range(CB1 // 16 + 1):
                sl = pl.ds(min(g * 16, CB1 - 16), 16)
                drs[b][sl] = di[b][sl]
            wb = wbuf[b]
            ea = eab[b]

            def edge(j, _):
                wb[j, pl.ds(0, 16)] = ea[pl.ds(j * ED, 16)]
                return 0
            lax.fori_loop(0, CB1, edge, 0)
            pltpu.async_copy(wb, acc_sh.at[drs[b]], sems[b], add=True)
            cn = c + 2
            cc = jnp.where(cn < nch, cn, 0)
            fire_idx(cc, b)
        return 0
    lax.fori_loop(0, nch // 2, body, 0)
    for b in range(2):
        wait_scatter(b)
        wait_idx(b)
    plsc.subcore_barrier()

    _copy_out_shared(acc_sh, acc_hbm, cid, sid)


def _s0(dst, eaf):
    return pl.kernel(
        _s0_body,
        out_type=jax.ShapeDtypeStruct((NC * N, 128), F32),
        mesh=plsc.VectorSubcoreMesh(**_MESH),
        scratch_types=[
            pltpu.VMEM((CB1,), I32),
            pltpu.VMEM((CB1,), I32),
            pltpu.VMEM((CB1,), I32),
            pltpu.VMEM((CB1,), I32),
            pltpu.VMEM((CB1 * ED,), F32),
            pltpu.VMEM((CB1 * ED,), F32),
            pltpu.VMEM((CB1, 128), F32),
            pltpu.VMEM((CB1, 128), F32),
            pltpu.VMEM((_SEGC, 128), F32),
            pltpu.VMEM_SHARED((N, 128), F32),
            pltpu.SemaphoreType.DMA,
            pltpu.SemaphoreType.DMA,
            pltpu.SemaphoreType.DMA,
            pltpu.SemaphoreType.DMA,
        ],
    )(dst, eaf)


def _s1h_body(t, soff_hbm, doff_hbm, dstr_hbm, tbl_hbm, ep_hbm, att_hbm,
              acc_hbm,
              si0, si1, di0, di1, dr0, dr1, drs0, drs1,
              srow0, srow1, drow0, drow1, epb, wbuf0, wbuf1,
              attb, zb, acc_sh,
              semi0, semi1, semg0, semg1, sems0, sems1, semp):
    cid = lax.axis_index("c")
    sid = lax.axis_index("s")
    lanes = lax.iota(I32, 16)
    mask0 = jnp.where(lanes == 0, 1.0, 0.0).astype(F32)
    head = 2 * t + cid

    si = (si0, si1)
    di = (di0, di1)
    dr = (dr0, dr1)
    drs = (drs0, drs1)
    srow = (srow0, srow1)
    drow = (drow0, drow1)
    wbuf = (wbuf0, wbuf1)
    semi = (semi0, semi1)
    semg = (semg0, semg1)
    sems = (sems0, sems1)

    pltpu.sync_copy(att_hbm, attb)
    att_vecs = [attb[pl.ds(head * HID + q * 16, 16)] for q in range(4)]

    for b in range(2):
        _zero_vmem(wbuf[b], 128)
    _zero_vmem(zb, 128)
    z16i = jnp.zeros((16,), I32)
    for b in range(2):
        for g in range(CB1 // 16 + 1):
            drs[b][pl.ds(min(g * 16, CB1 - 16), 16)] = z16i
    _zero_shared(zb, acc_sh, sid)
    plsc.subcore_barrier()

    nch = _EPT // CB1
    ebase = head * E + sid * _EPT
    rbase = sid * _EPT

    def fire_idx(c, b):
        bs = _m8(ebase + c * CB1)
        br = _m8(rbase + c * CB1)
        pltpu.async_copy(soff_hbm.at[pl.ds(bs, CB1)], si[b], semi[b])
        pltpu.async_copy(doff_hbm.at[pl.ds(bs, CB1)], di[b], semi[b])
        pltpu.async_copy(dstr_hbm.at[pl.ds(br, CB1)], dr[b], semi[b])

    def wait_idx(b):
        pltpu.make_async_copy(soff_hbm.at[pl.ds(0, CB1)], si[b], semi[b]).wait()
        pltpu.make_async_copy(doff_hbm.at[pl.ds(0, CB1)], di[b], semi[b]).wait()
        pltpu.make_async_copy(dstr_hbm.at[pl.ds(0, CB1)], dr[b], semi[b]).wait()

    def fire_gather(c, b):
        pltpu.async_copy(tbl_hbm.at[si[b]], srow[b], semg[b])
        pltpu.async_copy(tbl_hbm.at[di[b]], drow[b], semg[b])

    def wait_gather(b):
        pltpu.make_async_copy(tbl_hbm.at[si[b]], srow[b], semg[b]).wait()
        pltpu.make_async_copy(tbl_hbm.at[di[b]], drow[b], semg[b]).wait()

    def wait_scatter(b):
        pltpu.make_async_copy(acc_hbm.at[pl.ds(0, CB1)], wbuf[b],
                              sems[b]).wait()

    for b in range(2):
        pltpu.async_copy(wbuf[b], acc_sh.at[drs[b]], sems[b], add=True)
        fire_idx(b, b)
    for b in range(2):
        wait_idx(b)
        fire_gather(b, b)

    def body(k2, _):
        bp = _m8((ebase + 2 * k2 * CB1) // 2)
        cpe = pltpu.async_copy(ep_hbm.at[pl.ds(bp, CB1)], epb, semp)
        for b in range(2):
            c = 2 * k2 + b
            wait_scatter(b)
            wait_gather(b)
            if b == 0:
                cpe.wait()
            for g in range(CB1 // 16 + 1):
                sl = pl.ds(min(g * 16, CB1 - 16), 16)
                drs[b][sl] = dr[b][sl]
            cn = c + 2
            cc = jnp.where(cn < nch, cn, 0)
            fire_idx(cc, b)
            sr = srow[b]
            drr = drow[b]
            wb = wbuf[b]
            prow = b * (CB1 // 2)

            def quad(pq, _):
                for v in range(2):
                    p = 2 * pq + v
                    for u in range(2):
                        j = 2 * p + u
                        acc = _zeros16()
                        for q in range(4):
                            s = (sr[j, pl.ds(q * 16, 16)]
                                 + drr[j, pl.ds(64 + q * 16, 16)]
                                 + epb[prow + p, pl.ds(u * 64 + q * 16, 16)])
                            m = jnp.maximum(s, 0.2 * s)
                            acc = acc + m * att_vecs[q]
                        ex = jnp.exp(_allsum(acc))
                        for q in range(4):
                            sl = pl.ds(q * 16, 16)
                            wb[j, sl] = sr[j, sl] * ex
                        wb[j, pl.ds(64, 16)] = ex * mask0
                return 0
            lax.fori_loop(0, CB1 // 4, quad, 0)
            pltpu.async_copy(wb, acc_sh.at[drs[b]], sems[b], add=True)
            wait_idx(b)
            fire_gather(cc, b)
        return 0
    lax.fori_loop(0, nch // 2, body, 0)
    for b in range(2):
        wait_scatter(b)
        wait_gather(b)
    plsc.subcore_barrier()

    _copy_out_shared(acc_sh, acc_hbm, cid, sid)


def _s1h(t, soff, doff, dstr, tbl, ep, attf):
    return pl.kernel(
        functools.partial(_s1h_body, t),
        out_type=jax.ShapeDtypeStruct((NC * N, 128), F32),
        mesh=plsc.VectorSubcoreMesh(**_MESH),
        scratch_types=[
            pltpu.VMEM((CB1,), I32),
            pltpu.VMEM((CB1,), I32),
            pltpu.VMEM((CB1,), I32),
            pltpu.VMEM((CB1,), I32),
            pltpu.VMEM((CB1,), I32),
            pltpu.VMEM((CB1,), I32),
            pltpu.VMEM((CB1,), I32),
            pltpu.VMEM((CB1,), I32),
            pltpu.VMEM((CB1, 128), F32),
            pltpu.VMEM((CB1, 128), F32),
            pltpu.VMEM((CB1, 128), F32),
            pltpu.VMEM((CB1, 128), F32),
            pltpu.VMEM((CB1, 128), F32),
            pltpu.VMEM((CB1, 128), F32),
            pltpu.VMEM((CB1, 128), F32),
            pltpu.VMEM((256,), F32),
            pltpu.VMEM((_SEGC, 128), F32),
            pltpu.VMEM_SHARED((N, 128), F32),
            pltpu.SemaphoreType.DMA,
            pltpu.SemaphoreType.DMA,
            pltpu.SemaphoreType.DMA,
            pltpu.SemaphoreType.DMA,
            pltpu.SemaphoreType.DMA,
            pltpu.SemaphoreType.DMA,
            pltpu.SemaphoreType.DMA,
        ],
    )(soff, doff, dstr, tbl, ep, attf)


def _s2_body(src_hbm, dst_hbm, tbl_hbm, ep_hbm, att_hbm,
             acc_hbm,
             si0, si1, di0, di1, drs0, drs1,
             srow0, srow1, drow0, drow1, epb, wbuf0, wbuf1,
             attb, zb, acc_sh,
             semi0, semi1, semg0, semg1, sems0, sems1, semp):
    cid = lax.axis_index("c")
    sid = lax.axis_index("s")
    wid = cid * NS + sid
    lanes = lax.iota(I32, 16)
    mask0 = jnp.where(lanes == 0, 1.0, 0.0).astype(F32)

    si = (si0, si1)
    di = (di0, di1)
    drs = (drs0, drs1)
    srow = (srow0, srow1)
    drow = (drow0, drow1)
    wbuf = (wbuf0, wbuf1)
    semi = (semi0, semi1)
    semg = (semg0, semg1)
    sems = (sems0, sems1)

    pltpu.sync_copy(att_hbm, attb)
    att_vecs = [attb[pl.ds(q * 16, 16)] for q in range(4)]

    for b in range(2):
        _zero_vmem(wbuf[b], 128)
    _zero_vmem(zb, 128)
    z16i = jnp.zeros((16,), I32)
    for b in range(2):
        for g in range(CB1 // 16 + 1):
            drs[b][pl.ds(min(g * 16, CB1 - 16), 16)] = z16i
    _zero_shared(zb, acc_sh, sid)
    plsc.subcore_barrier()

    nch = _EPW // CB1
    ebase = wid * _EPW

    def fire_idx(c, b):
        bs = _m8(ebase + c * CB1)
        pltpu.async_copy(src_hbm.at[pl.ds(bs, CB1)], si[b], semi[b])
        pltpu.async_copy(dst_hbm.at[pl.ds(bs, CB1)], di[b], semi[b])

    def wait_idx(b):
        pltpu.make_async_copy(src_hbm.at[pl.ds(0, CB1)], si[b], semi[b]).wait()
        pltpu.make_async_copy(dst_hbm.at[pl.ds(0, CB1)], di[b], semi[b]).wait()

    def fire_gather(c, b):
        pltpu.async_copy(tbl_hbm.at[si[b]], srow[b], semg[b])
        pltpu.async_copy(tbl_hbm.at[di[b]], drow[b], semg[b])

    def wait_gather(b):
        pltpu.make_async_copy(tbl_hbm.at[si[b]], srow[b], semg[b]).wait()
        pltpu.make_async_copy(tbl_hbm.at[di[b]], drow[b], semg[b]).wait()

    def wait_scatter(b):
        pltpu.make_async_copy(acc_hbm.at[pl.ds(0, CB1)], wbuf[b],
                              sems[b]).wait()

    for b in range(2):
        pltpu.async_copy(wbuf[b], acc_sh.at[drs[b]], sems[b], add=True)
        fire_idx(b, b)
    for b in range(2):
        wait_idx(b)
        fire_gather(b, b)

    def body(k2, _):
        bp = _m8((ebase + 2 * k2 * CB1) // 2)
        cpe = pltpu.async_copy(ep_hbm.at[pl.ds(bp, CB1)], epb, semp)
        for b in range(2):
            c = 2 * k2 + b
            wait_scatter(b)
            wait_gather(b)
            if b == 0:
                cpe.wait()
            for g in range(CB1 // 16 + 1):
                sl = pl.ds(min(g * 16, CB1 - 16), 16)
                drs[b][sl] = di[b][sl]
            cn = c + 2
            cc = jnp.where(cn < nch, cn, 0)
            fire_idx(cc, b)
            sr = srow[b]
            drr = drow[b]
            wb = wbuf[b]
            prow = b * (CB1 // 2)

            def quad(pq, _):
                for v in range(2):
                    p = 2 * pq + v
                    for u in range(2):
                        j = 2 * p + u
                        acc = _zeros16()
                        for q in range(4):
                            s = (sr[j, pl.ds(q * 16, 16)]
                                 + drr[j, pl.ds(64 + q * 16, 16)]
                                 + epb[prow + p, pl.ds(u * 64 + q * 16, 16)])
                            m = jnp.maximum(s, 0.2 * s)
                            acc = acc + m * att_vecs[q]
                        ex = jnp.exp(_allsum(acc))
                        for q in range(4):
                            sl = pl.ds(q * 16, 16)
                            wb[j, sl] = sr[j, sl] * ex
                        wb[j, pl.ds(64, 16)] = ex * mask0
                return 0
            lax.fori_loop(0, CB1 // 4, quad, 0)
            pltpu.async_copy(wb, acc_sh.at[drs[b]], sems[b], add=True)
            wait_idx(b)
            fire_gather(cc, b)
        return 0
    lax.fori_loop(0, nch // 2, body, 0)
    for b in range(2):
        wait_scatter(b)
        wait_gather(b)
    plsc.subcore_barrier()

    _copy_out_shared(acc_sh, acc_hbm, cid, sid)


def _s2(src, dst, tbl, ep, attf):
    return pl.kernel(
        _s2_body,
        out_type=jax.ShapeDtypeStruct((NC * N, 128), F32),
        mesh=plsc.VectorSubcoreMesh(**_MESH),
        scratch_types=[
            pltpu.VMEM((CB1,), I32),
            pltpu.VMEM((CB1,), I32),
            pltpu.VMEM((CB1,), I32),
            pltpu.VMEM((CB1,), I32),
            pltpu.VMEM((CB1,), I32),
            pltpu.VMEM((CB1,), I32),
            pltpu.VMEM((CB1, 128), F32),
            pltpu.VMEM((CB1, 128), F32),
            pltpu.VMEM((CB1, 128), F32),
            pltpu.VMEM((CB1, 128), F32),
            pltpu.VMEM((CB1, 128), F32),
            pltpu.VMEM((CB1, 128), F32),
            pltpu.VMEM((CB1, 128), F32),
            pltpu.VMEM((64,), F32),
            pltpu.VMEM((_SEGC, 128), F32),
            pltpu.VMEM_SHARED((N, 128), F32),
            pltpu.SemaphoreType.DMA,
            pltpu.SemaphoreType.DMA,
            pltpu.SemaphoreType.DMA,
            pltpu.SemaphoreType.DMA,
            pltpu.SemaphoreType.DMA,
            pltpu.SemaphoreType.DMA,
            pltpu.SemaphoreType.DMA,
        ],
    )(src, dst, tbl, ep, attf)


def _s3_body(src_hbm, dst_hbm, g_hbm, eec_hbm, cw_hbm,
             lg_hbm,
             si0, si1, di0, di1, srow0, srow1, drow0, drow1, ecb0, ecb1,
             wb_, lbuf, sem0, semi0, semi1, semg0, semg1):
    cid = lax.axis_index("c")
    sid = lax.axis_index("s")
    wid = cid * NS + sid
    lanes = lax.iota(I32, 16)

    si = (si0, si1)
    di = (di0, di1)
    srow = (srow0, srow1)
    drow = (drow0, drow1)
    ecb = (ecb0, ecb1)
    semi = (semi0, semi1)
    semg = (semg0, semg1)

    pltpu.sync_copy(cw_hbm, wb_)
    w_vecs = [wb_[pl.ds(q * 16, 16)] for q in range(4)]
    cb_vec = wb_[pl.ds(64, 16)]

    nch = _EPW // CB
    ebase = wid * _EPW

    def fire_idx(c, b):
        bs = _m8(ebase + c * CB)
        pltpu.async_copy(src_hbm.at[pl.ds(bs, CB)], si[b], semi[b])
        pltpu.async_copy(dst_hbm.at[pl.ds(bs, CB)], di[b], semi[b])
        pltpu.async_copy(eec_hbm.at[pl.ds(_m8(bs // 2), CB // 2)],
                         ecb[b], semi[b])

    def wait_idx(b):
        pltpu.make_async_copy(src_hbm.at[pl.ds(0, CB)], si[b], semi[b]).wait()
        pltpu.make_async_copy(dst_hbm.at[pl.ds(0, CB)], di[b], semi[b]).wait()
        pltpu.make_async_copy(eec_hbm.at[pl.ds(0, CB // 2)], ecb[b],
                              semi[b]).wait()

    def fire_gather(b):
        pltpu.async_copy(g_hbm.at[si[b]], srow[b], semg[b])
        pltpu.async_copy(g_hbm.at[di[b]], drow[b], semg[b])

    def wait_gather(b):
        pltpu.make_async_copy(g_hbm.at[pl.ds(0, CB)], srow[b], semg[b]).wait()
        pltpu.make_async_copy(g_hbm.at[pl.ds(0, CB)], drow[b], semg[b]).wait()

    def compute_write(c, b):
        sr = srow[b]
        drr = drow[b]
        ec = ecb[b]

        def group(g, _):
            lvec = cb_vec
            for jj in range(16):
                j = g * 16 + jj
                acc = _zeros16()
                for q in range(4):
                    z = (sr[j, pl.ds(q * 16, 16)]
                         + drr[j, pl.ds(64 + q * 16, 16)]
                         + ec[(g * 16 + jj) // 2,
                              pl.ds((jj % 2) * 64 + q * 16, 16)])
                    z = jnp.maximum(z, 0.0)
                    acc = acc + z * w_vecs[q]
                a = _allsum(acc)
                mj = jnp.where(lanes == jj, 1.0, 0.0).astype(F32)
                lvec = lvec + a * mj
            lbuf[pl.ds(g * 16, 16)] = lvec
            return 0
        lax.fori_loop(0, CB // 16, group, 0)
        pltpu.sync_copy(lbuf, lg_hbm.at[pl.ds(_m8(ebase + c * CB), CB)])

    fire_idx(0, 0)
    fire_idx(1, 1)
    wait_idx(0)
    fire_gather(0)
    wait_idx(1)
    fire_gather(1)
    wait_gather(0)
    compute_write(0, 0)
    fire_idx(2, 0)

    def body(k2, _):
        ca = 1 + 2 * k2
        wait_gather(1)
        wait_idx(0)
        fire_gather(0)
        compute_write(ca, 1)
        cn = ca + 2
        fire_idx(jnp.where(cn < nch, cn, 0), 1)
        wait_gather(0)
        wait_idx(1)
        fire_gather(1)
        compute_write(ca + 1, 0)
        cn2 = ca + 3
        fire_idx(jnp.where(cn2 < nch, cn2, 0), 0)
        return 0
    lax.fori_loop(0, (nch - 1) // 2, body, 0)
    wait_gather(1)
    wait_idx(0)


def _s3(src, dst, g, eec, cwm):
    return pl.kernel(
        _s3_body,
        out_type=jax.ShapeDtypeStruct((E,), F32),
        mesh=plsc.VectorSubcoreMesh(**_MESH),
        scratch_types=[
            pltpu.VMEM((CB,), I32),
            pltpu.VMEM((CB,), I32),
            pltpu.VMEM((CB,), I32),
            pltpu.VMEM((CB,), I32),
            pltpu.VMEM((CB, 128), F32),
            pltpu.VMEM((CB, 128), F32),
            pltpu.VMEM((CB, 128), F32),
            pltpu.VMEM((CB, 128), F32),
            pltpu.VMEM((CB // 2, 128), F32),
            pltpu.VMEM((CB // 2, 128), F32),
            pltpu.VMEM((80,), F32),
            pltpu.VMEM((CB,), F32),
            pltpu.SemaphoreType.DMA,
            pltpu.SemaphoreType.DMA,
            pltpu.SemaphoreType.DMA,
            pltpu.SemaphoreType.DMA,
            pltpu.SemaphoreType.DMA,
        ],
    )(src, dst, g, eec, cwm)


def _ep1_body(acca_ref, accb_ref, s0_ref, xlr_ref,
              ewT_ref, s16_ref, s64_ref, attm_ref, bias_ref,
              l2lwT_ref, l2lb_ref, l2rwT_ref, l2rb_ref,
              c2_ref, la_ref):
    s0s = s0_ref[0] + s0_ref[1]
    ea = s0s[:, :ED]
    cntv = jnp.dot(s0s, s16_ref[...], preferred_element_type=F32)
    la = ea / jnp.maximum(cntv, 1.0)
    la_ref[...] = la
    lp = jnp.dot(la, ewT_ref[...], preferred_element_type=F32)

    accs = [acca_ref[0], acca_ref[1], accb_ref[0], accb_ref[1]]
    xls = []
    mls = []
    for h in range(4):
        xlh = xlr_ref[h][:, :HID]
        xrh = xlr_ref[h][:, HID:]
        mm = xlh + xrh + lp[:, h * HID:(h + 1) * HID]
        xls.append(xlh)
        mls.append(jnp.maximum(mm, 0.2 * mm))
    al = sum(jnp.dot(mls[h], attm_ref[h * HID:(h + 1) * HID, :],
                     preferred_element_type=F32) for h in range(4))
    exl = jnp.exp(al)
    xl2v = l2lb_ref[...]
    xr2v = l2rb_ref[...]
    for h in range(4):
        exh = exl[:, h:h + 1]
        denh = jnp.dot(accs[h], s64_ref[...], preferred_element_type=F32)
        num = accs[h][:, :HID] + exh * xls[h]
        hv = num / (denh + exh + 1e-16) + bias_ref[:, h * HID:(h + 1) * HID]
        hv = jnp.where(hv > 0, hv, jnp.exp(jnp.minimum(hv, 0.0)) - 1.0)
        xl2v = xl2v + jnp.dot(hv, l2lwT_ref[h * HID:(h + 1) * HID, :],
                              preferred_element_type=F32)
        xr2v = xr2v + jnp.dot(hv, l2rwT_ref[h * HID:(h + 1) * HID, :],
                              preferred_element_type=F32)
    c2_ref[...] = jnp.concatenate([xl2v, xr2v], axis=1)


def _ep1(acca, accb, s0acc, xlr, ewT, s16, s64, attm, bias,
         l2lwT, l2lb, l2rwT, l2rb):
    def full(shape):
        return pl.BlockSpec(shape, lambda i, _n=len(shape): (0,) * _n)
    return pl.pallas_call(
        _ep1_body,
        grid=(N // BN,),
        in_specs=[
            pl.BlockSpec((NC, BN, 128), lambda i: (0, i, 0)),
            pl.BlockSpec((NC, BN, 128), lambda i: (0, i, 0)),
            pl.BlockSpec((NC, BN, 128), lambda i: (0, i, 0)),
            pl.BlockSpec((4, BN, 128), lambda i: (0, i, 0)),
            full((ED, 256)),
            full((128, 1)),
            full((128, 1)),
            full((256, 4)),
            full((1, 256)),
            full((256, HID)),
            full((1, HID)),
            full((256, HID)),
            full((1, HID)),
        ],
        out_specs=[pl.BlockSpec((BN, 128), lambda i: (i, 0)),
                   pl.BlockSpec((BN, ED), lambda i: (i, 0))],
        out_shape=[jax.ShapeDtypeStruct((N, 128), F32),
                   jax.ShapeDtypeStruct((N, ED), F32)],
    )(acca, accb, s0acc, xlr, ewT, s16, s64, attm, bias,
      l2lwT, l2lb, l2rwT, l2rb)


def _ep2_body(acc_ref, c2_ref, la_ref,
              ew2T_ref, att2_ref, s64_ref, bias_ref, wsT_ref, wdT_ref,
              g_ref):
    accs = acc_ref[0] + acc_ref[1]
    den = jnp.dot(accs, s64_ref[...], preferred_element_type=F32)
    lp2 = jnp.dot(la_ref[...], ew2T_ref[...], preferred_element_type=F32)
    xl2 = c2_ref[:, :HID]
    xr2 = c2_ref[:, HID:]
    mm = xl2 + xr2 + lp2
    ml = jnp.maximum(mm, 0.2 * mm)
    al = jnp.dot(ml, att2_ref[...], preferred_element_type=F32)
    ex = jnp.exp(al)
    h2 = (accs[:, :HID] + ex * xl2) / (den + ex + 1e-16) + bias_ref[...]
    h2 = jnp.where(h2 > 0, h2, jnp.exp(jnp.minimum(h2, 0.0)) - 1.0)
    g_ref[...] = jnp.concatenate(
        [jnp.dot(h2, wsT_ref[...], preferred_element_type=F32),
         jnp.dot(h2, wdT_ref[...], preferred_element_type=F32)], axis=1)


def _ep2(acc, c2, la, ew2T, att2, s64, bias, wsT, wdT):
    def full(shape):
        return pl.BlockSpec(shape, lambda i, _n=len(shape): (0,) * _n)
    return pl.pallas_call(
        _ep2_body,
        grid=(N // BN,),
        in_specs=[
            pl.BlockSpec((NC, BN, 128), lambda i: (0, i, 0)),
            pl.BlockSpec((BN, 128), lambda i: (i, 0)),
            pl.BlockSpec((BN, ED), lambda i: (i, 0)),
            full((ED, HID)),
            full((HID, 1)),
            full((128, 1)),
            full((1, HID)),
            full((HID, HID)),
            full((HID, HID)),
        ],
        out_specs=pl.BlockSpec((BN, 128), lambda i: (i, 0)),
        out_shape=jax.ShapeDtypeStruct((N, 128), F32),
    )(acc, c2, la, ew2T, att2, s64, bias, wsT, wdT)


_S16 = np.zeros((128, 1), np.float32)
_S16[16, 0] = 1.0
_S64 = np.zeros((128, 1), np.float32)
_S64[64, 0] = 1.0


def kernel(x, edge_index, edge_attr, l1_lw, l1_lb, l1_rw, l1_rb, l1_ew,
           l1_att, l1_bias, l2_lw, l2_lb, l2_rw, l2_rb, l2_ew, l2_att,
           l2_bias, c1_w, c1_b, c2_w, c2_b):
    src = edge_index[0]
    dst = edge_index[1]
    eae = edge_attr[0::2]
    eao = edge_attr[1::2]
    eaf = edge_attr.reshape(-1)

    def hsplit(w):
        return w.reshape(w.shape[0], 4, HID).transpose(1, 0, 2)

    tbl1 = _t1(x, hsplit(l1_lw.T), l1_lb.reshape(4, 1, HID),
               hsplit(l1_rw.T), l1_rb.reshape(4, 1, HID))
    ep1 = _t2a(eae, eao, hsplit(l1_ew.T))
    ep2, eec = _t2b(eae, eao, l2_ew.T, c1_w[:, 128:].T, c1_b[None])

    s0acc = _s0(dst, eaf)

    tblf = tbl1.reshape(4 * N, 128)
    epf = ep1.reshape(4 * (E // 2), 128)
    attm1 = l1_att.reshape(-1)
    hoff = (jnp.arange(4, dtype=I32) * N)[:, None]
    srcoff = (src[None, :] + hoff).reshape(-1)
    dstoff = (dst[None, :] + hoff).reshape(-1)
    acca = _s1h(0, srcoff, dstoff, dst, tblf, epf, attm1)
    accb = _s1h(1, srcoff, dstoff, dst, tblf, epf, attm1)

    attmask = jnp.zeros((256, 4), F32).at[
        jnp.arange(256), jnp.arange(256) // 64].set(l1_att.reshape(-1))
    c2tbl, la = _ep1(
        acca.reshape(NC, N, 128), accb.reshape(NC, N, 128),
        s0acc.reshape(NC, N, 128), tbl1,
        l1_ew.T, jnp.asarray(_S16), jnp.asarray(_S64),
        attmask, l1_bias[None],
        l2_lw.T, l2_lb[None], l2_rw.T, l2_rb[None])

    acc2 = _s2(src, dst, c2tbl, ep2, l2_att.reshape(-1))

    gtbl = _ep2(acc2.reshape(NC, N, 128), c2tbl, la,
                l2_ew.T, l2_att.reshape(HID, 1), jnp.asarray(_S64),
                l2_bias[None], c1_w[:, :64].T, c1_w[:, 64:128].T)

    cwm = jnp.concatenate([c2_w.reshape(-1),
                           jnp.broadcast_to(c2_b, (16,))])
    logits = _s3(src, dst, gtbl, eec, cwm)
    return logits


# speedup vs baseline: 13.5556x; 1.0015x over previous
import functools

import jax
import jax.numpy as jnp
import numpy as np
from jax import lax
from jax.experimental import pallas as pl
from jax.experimental.pallas import tpu as pltpu
from jax.experimental.pallas import tpu_sc as plsc

N = 10000
E = 320000
ND = 128
ED = 16
HID = 64
NC = 2
NS = 16
CB = 80
CB1 = 40
BN = 1000
BEP = 2000

F32 = jnp.float32
I32 = jnp.int32

_MESH = dict(core_axis_name="c", subcore_axis_name="s")

_EPW = E // (NC * NS)
_EPT = E // NS
_SEG = 624
_SEGC = 104
_TAIL = N - NS * _SEG

_DNUMS = lax.GatherDimensionNumbers(
    offset_dims=(), collapsed_slice_dims=(0,), start_index_map=(0,))


def _allsum(v):
    lanes = lax.iota(I32, 16)
    for k in range(4):
        p = lax.bitwise_xor(lanes, 1 << k)
        v = v + lax.gather(v, p[:, None], _DNUMS, slice_sizes=(1,),
                           mode=lax.GatherScatterMode.PROMISE_IN_BOUNDS)
    return v


def _zeros16():
    return jnp.zeros((16,), F32)


def _m8(v):
    return pl.multiple_of(v, 8)


def _zero_vmem(buf, w):
    z16 = _zeros16()

    def zr(i, _):
        for q in range(w // 16):
            buf[i, pl.ds(q * 16, 16)] = z16
        return 0
    lax.fori_loop(0, buf.shape[0], zr, 0)


def _zero_shared(zb, sh, sid):
    for t in range(_SEG // _SEGC):
        pltpu.sync_copy(zb, sh.at[pl.ds(_m8(sid * _SEG + t * _SEGC), _SEGC)])

    @pl.when(sid == NS - 1)
    def _():
        pltpu.sync_copy(zb.at[pl.ds(0, _TAIL)], sh.at[pl.ds(NS * _SEG, _TAIL)])


def _copy_out_shared(sh, hbm, cid, sid):
    pltpu.sync_copy(sh.at[pl.ds(_m8(sid * _SEG), _SEG)],
                    hbm.at[pl.ds(_m8(cid * N + sid * _SEG), _SEG)])

    @pl.when(sid == NS - 1)
    def _():
        pltpu.sync_copy(sh.at[pl.ds(NS * _SEG, _TAIL)],
                        hbm.at[pl.ds(_m8(cid * N + NS * _SEG), _TAIL)])


def _t1_body(x_ref, lwT_ref, lb_ref, rwT_ref, rb_ref, c_ref):
    xv = x_ref[...]
    xl = jnp.dot(xv, lwT_ref[0], preferred_element_type=F32) + lb_ref[0]
    xr = jnp.dot(xv, rwT_ref[0], preferred_element_type=F32) + rb_ref[0]
    c_ref[...] = jnp.concatenate([xl, xr], axis=1)[None]


def _t1(x, lwT, lb, rwT, rb):
    return pl.pallas_call(
        _t1_body,
        grid=(4, N // BN),
        in_specs=[
            pl.BlockSpec((BN, ND), lambda c, i: (i, 0)),
            pl.BlockSpec((1, ND, HID), lambda c, i: (c, 0, 0)),
            pl.BlockSpec((1, 1, HID), lambda c, i: (c, 0, 0)),
            pl.BlockSpec((1, ND, HID), lambda c, i: (c, 0, 0)),
            pl.BlockSpec((1, 1, HID), lambda c, i: (c, 0, 0)),
        ],
        out_specs=pl.BlockSpec((1, BN, 128), lambda c, i: (c, i, 0)),
        out_shape=jax.ShapeDtypeStruct((4, N, 128), F32),
    )(x, lwT, lb, rwT, rb)


def _t2a_body(eae_ref, eao_ref, ewT_ref, ep_ref):
    pe = jnp.dot(eae_ref[...], ewT_ref[0], preferred_element_type=F32)
    po = jnp.dot(eao_ref[...], ewT_ref[0], preferred_element_type=F32)
    ep_ref[...] = jnp.concatenate([pe, po], axis=1)[None]


def _t2a(eae, eao, ewT):
    return pl.pallas_call(
        _t2a_body,
        grid=(4, (E // 2) // BEP),
        in_specs=[
            pl.BlockSpec((BEP, ED), lambda c, i: (i, 0)),
            pl.BlockSpec((BEP, ED), lambda c, i: (i, 0)),
            pl.BlockSpec((1, ED, HID), lambda c, i: (c, 0, 0)),
        ],
        out_specs=pl.BlockSpec((1, BEP, 128), lambda c, i: (c, i, 0)),
        out_shape=jax.ShapeDtypeStruct((4, E // 2, 128), F32),
    )(eae, eao, ewT)


def _t2b_body(eae_ref, eao_ref, w2T_ref, weT_ref, cb_ref, ep2_ref, eec_ref):
    eav = eae_ref[...]
    eov = eao_ref[...]
    ep2_ref[...] = jnp.concatenate(
        [jnp.dot(eav, w2T_ref[...], preferred_element_type=F32),
         jnp.dot(eov, w2T_ref[...], preferred_element_type=F32)], axis=1)
    eec_ref[...] = jnp.concatenate(
        [jnp.dot(eav, weT_ref[...], preferred_element_type=F32) + cb_ref[...],
         jnp.dot(eov, weT_ref[...], preferred_element_type=F32) + cb_ref[...]],
        axis=1)


def _t2b(eae, eao, w2T, weT, cb):
    return pl.pallas_call(
        _t2b_body,
        grid=((E // 2) // BEP,),
        in_specs=[
            pl.BlockSpec((BEP, ED), lambda i: (i, 0)),
            pl.BlockSpec((BEP, ED), lambda i: (i, 0)),
            pl.BlockSpec((ED, HID), lambda i: (0, 0)),
            pl.BlockSpec((ED, HID), lambda i: (0, 0)),
            pl.BlockSpec((1, HID), lambda i: (0, 0)),
        ],
        out_specs=[pl.BlockSpec((BEP, 128), lambda i: (i, 0))] * 2,
        out_shape=[jax.ShapeDtypeStruct((E // 2, 128), F32)] * 2,
    )(eae, eao, w2T, weT, cb)


def _s0_body(dst_hbm, ea_hbm, acc_hbm,
             di0, di1, drs0, drs1, eab0, eab1, wbuf0, wbuf1, zb, acc_sh,
             semi0, semi1, sems0, sems1):
    cid = lax.axis_index("c")
    sid = lax.axis_index("s")
    wid = cid * NS + sid
    lanes = lax.iota(I32, 16)
    mask0 = jnp.where(lanes == 0, 1.0, 0.0).astype(F32)

    di = (di0, di1)
    drs = (drs0, drs1)
    eab = (eab0, eab1)
    wbuf = (wbuf0, wbuf1)
    semi = (semi0, semi1)
    sems = (sems0, sems1)

    for b in range(2):
        _zero_vmem(wbuf[b], 128)

        def fill(i, _, _b=b):
            wbuf[_b][i, pl.ds(16, 16)] = mask0
            return 0
        lax.fori_loop(0, CB1, fill, 0)
    _zero_vmem(zb, 128)
    z16i = jnp.zeros((16,), I32)
    for b in range(2):
        for g in range(CB1 // 16 + 1):
            drs[b][pl.ds(min(g * 16, CB1 - 16), 16)] = z16i
    _zero_shared(zb, acc_sh, sid)
    plsc.subcore_barrier()

    nch = _EPW // CB1
    ebase = wid * _EPW

    def fire_idx(c, b):
        bs = _m8(ebase + c * CB1)
        pltpu.async_copy(dst_hbm.at[pl.ds(bs, CB1)], di[b], semi[b])
        pltpu.async_copy(ea_hbm.at[pl.ds(_m8(bs * ED), CB1 * ED)],
                         eab[b], semi[b])

    def wait_idx(b):
        pltpu.make_async_copy(dst_hbm.at[pl.ds(0, CB1)], di[b],
                              semi[b]).wait()
        pltpu.make_async_copy(ea_hbm.at[pl.ds(0, CB1 * ED)], eab[b],
                              semi[b]).wait()

    def wait_scatter(b):
        pltpu.make_async_copy(acc_hbm.at[pl.ds(0, CB1)], wbuf[b],
                              sems[b]).wait()

    for b in range(2):
        pltpu.async_copy(wbuf[b], acc_sh.at[drs[b]], sems[b], add=True)
        fire_idx(b, b)

    def body(k2, _):
        for b in range(2):
            c = 2 * k2 + b
            wait_scatter(b)
            wait_idx(b)
            for g in range(CB1 // 16 + 1):
                sl = pl.ds(min(g * 16, CB1 - 16), 16)
                drs[b][sl] = di[b][sl]
            wb = wbuf[b]
            ea = eab[b]

            def edge(j, _):
                wb[j, pl.ds(0, 16)] = ea[pl.ds(j * ED, 16)]
                return 0
            lax.fori_loop(0, CB1, edge, 0)
            pltpu.async_copy(wb, acc_sh.at[drs[b]], sems[b], add=True)
            cn = c + 2
            cc = jnp.where(cn < nch, cn, 0)
            fire_idx(cc, b)
        return 0
    lax.fori_loop(0, nch // 2, body, 0)
    for b in range(2):
        wait_scatter(b)
        wait_idx(b)
    plsc.subcore_barrier()

    _copy_out_shared(acc_sh, acc_hbm, cid, sid)


def _s0(dst, eaf):
    return pl.kernel(
        _s0_body,
        out_type=jax.ShapeDtypeStruct((NC * N, 128), F32),
        mesh=plsc.VectorSubcoreMesh(**_MESH),
        scratch_types=[
            pltpu.VMEM((CB1,), I32),
            pltpu.VMEM((CB1,), I32),
            pltpu.VMEM((CB1,), I32),
            pltpu.VMEM((CB1,), I32),
            pltpu.VMEM((CB1 * ED,), F32),
            pltpu.VMEM((CB1 * ED,), F32),
            pltpu.VMEM((CB1, 128), F32),
            pltpu.VMEM((CB1, 128), F32),
            pltpu.VMEM((_SEGC, 128), F32),
            pltpu.VMEM_SHARED((N, 128), F32),
            pltpu.SemaphoreType.DMA,
            pltpu.SemaphoreType.DMA,
            pltpu.SemaphoreType.DMA,
            pltpu.SemaphoreType.DMA,
        ],
    )(dst, eaf)


def _s1h_body(t, soff_hbm, doff_hbm, tbl_hbm, ep_hbm, att_hbm,
              acc_hbm,
              si0, si1, di0, di1, drs0, drs1,
              srow0, srow1, drow0, drow1, epb, wbuf0, wbuf1,
              attb, zb, acc_sh,
              semi0, semi1, semg0, semg1, sems0, sems1, semp):
    cid = lax.axis_index("c")
    sid = lax.axis_index("s")
    lanes = lax.iota(I32, 16)
    mask0 = jnp.where(lanes == 0, 1.0, 0.0).astype(F32)
    head = 2 * t + cid

    si = (si0, si1)
    di = (di0, di1)
    drs = (drs0, drs1)
    srow = (srow0, srow1)
    drow = (drow0, drow1)
    wbuf = (wbuf0, wbuf1)
    semi = (semi0, semi1)
    semg = (semg0, semg1)
    sems = (sems0, sems1)

    pltpu.sync_copy(att_hbm, attb)
    att_vecs = [attb[pl.ds(head * HID + q * 16, 16)] for q in range(4)]

    for b in range(2):
        _zero_vmem(wbuf[b], 128)
    _zero_vmem(zb, 128)
    z16i = jnp.zeros((16,), I32)
    for b in range(2):
        for g in range(CB1 // 16 + 1):
            drs[b][pl.ds(min(g * 16, CB1 - 16), 16)] = z16i
    _zero_shared(zb, acc_sh, sid)
    plsc.subcore_barrier()

    nch = _EPT // CB1
    ebase = head * E + sid * _EPT
    ioff = head * N

    def fire_idx(c, b):
        bs = _m8(ebase + c * CB1)
        pltpu.async_copy(soff_hbm.at[pl.ds(bs, CB1)], si[b], semi[b])
        pltpu.async_copy(doff_hbm.at[pl.ds(bs, CB1)], di[b], semi[b])

    def wait_idx(b):
        pltpu.make_async_copy(soff_hbm.at[pl.ds(0, CB1)], si[b], semi[b]).wait()
        pltpu.make_async_copy(doff_hbm.at[pl.ds(0, CB1)], di[b], semi[b]).wait()

    def fire_gather(c, b):
        pltpu.async_copy(tbl_hbm.at[si[b]], srow[b], semg[b])
        pltpu.async_copy(tbl_hbm.at[di[b]], drow[b], semg[b])

    def wait_gather(b):
        pltpu.make_async_copy(tbl_hbm.at[si[b]], srow[b], semg[b]).wait()
        pltpu.make_async_copy(tbl_hbm.at[di[b]], drow[b], semg[b]).wait()

    def wait_scatter(b):
        pltpu.make_async_copy(acc_hbm.at[pl.ds(0, CB1)], wbuf[b],
                              sems[b]).wait()

    for b in range(2):
        pltpu.async_copy(wbuf[b], acc_sh.at[drs[b]], sems[b], add=True)
        fire_idx(b, b)
    for b in range(2):
        wait_idx(b)
        fire_gather(b, b)

    def body(k2, _):
        bp = _m8((ebase + 2 * k2 * CB1) // 2)
        cpe = pltpu.async_copy(ep_hbm.at[pl.ds(bp, CB1)], epb, semp)
        for b in range(2):
            c = 2 * k2 + b
            wait_scatter(b)
            wait_gather(b)
            if b == 0:
                cpe.wait()
            for g in range(CB1 // 16 + 1):
                sl = pl.ds(min(g * 16, CB1 - 16), 16)
                drs[b][sl] = di[b][sl] - ioff
            cn = c + 2
            cc = jnp.where(cn < nch, cn, 0)
            fire_idx(cc, b)
            sr = srow[b]
            drr = drow[b]
            wb = wbuf[b]
            prow = b * (CB1 // 2)

            def quad(pq, _):
                for v in range(2):
                    p = 2 * pq + v
                    for u in range(2):
                        j = 2 * p + u
                        acc = _zeros16()
                        for q in range(4):
                            s = (sr[j, pl.ds(q * 16, 16)]
                                 + drr[j, pl.ds(64 + q * 16, 16)]
                                 + epb[prow + p, pl.ds(u * 64 + q * 16, 16)])
                            m = jnp.maximum(s, 0.2 * s)
                            acc = acc + m * att_vecs[q]
                        ex = jnp.exp(_allsum(acc))
                        for q in range(4):
                            sl = pl.ds(q * 16, 16)
                            wb[j, sl] = sr[j, sl] * ex
                        wb[j, pl.ds(64, 16)] = ex * mask0
                return 0
            lax.fori_loop(0, CB1 // 4, quad, 0)
            pltpu.async_copy(wb, acc_sh.at[drs[b]], sems[b], add=True)
            wait_idx(b)
            fire_gather(cc, b)
        return 0
    lax.fori_loop(0, nch // 2, body, 0)
    for b in range(2):
        wait_scatter(b)
        wait_gather(b)
    plsc.subcore_barrier()

    _copy_out_shared(acc_sh, acc_hbm, cid, sid)


def _s1h(t, soff, doff, tbl, ep, attf):
    return pl.kernel(
        functools.partial(_s1h_body, t),
        out_type=jax.ShapeDtypeStruct((NC * N, 128), F32),
        mesh=plsc.VectorSubcoreMesh(**_MESH),
        scratch_types=[
            pltpu.VMEM((CB1,), I32),
            pltpu.VMEM((CB1,), I32),
            pltpu.VMEM((CB1,), I32),
            pltpu.VMEM((CB1,), I32),
            pltpu.VMEM((CB1,), I32),
            pltpu.VMEM((CB1,), I32),
            pltpu.VMEM((CB1, 128), F32),
            pltpu.VMEM((CB1, 128), F32),
            pltpu.VMEM((CB1, 128), F32),
            pltpu.VMEM((CB1, 128), F32),
            pltpu.VMEM((CB1, 128), F32),
            pltpu.VMEM((CB1, 128), F32),
            pltpu.VMEM((CB1, 128), F32),
            pltpu.VMEM((256,), F32),
            pltpu.VMEM((_SEGC, 128), F32),
            pltpu.VMEM_SHARED((N, 128), F32),
            pltpu.SemaphoreType.DMA,
            pltpu.SemaphoreType.DMA,
            pltpu.SemaphoreType.DMA,
            pltpu.SemaphoreType.DMA,
            pltpu.SemaphoreType.DMA,
            pltpu.SemaphoreType.DMA,
            pltpu.SemaphoreType.DMA,
        ],
    )(soff, doff, tbl, ep, attf)


def _s2_body(src_hbm, dst_hbm, tbl_hbm, ep_hbm, att_hbm,
             acc_hbm,
             si0, si1, di0, di1, drs0, drs1,
             srow0, srow1, drow0, drow1, epb, wbuf0, wbuf1,
             attb, zb, acc_sh,
             semi0, semi1, semg0, semg1, sems0, sems1, semp):
    cid = lax.axis_index("c")
    sid = lax.axis_index("s")
    wid = cid * NS + sid
    lanes = lax.iota(I32, 16)
    mask0 = jnp.where(lanes == 0, 1.0, 0.0).astype(F32)

    si = (si0, si1)
    di = (di0, di1)
    drs = (drs0, drs1)
    srow = (srow0, srow1)
    drow = (drow0, drow1)
    wbuf = (wbuf0, wbuf1)
    semi = (semi0, semi1)
    semg = (semg0, semg1)
    sems = (sems0, sems1)

    pltpu.sync_copy(att_hbm, attb)
    att_vecs = [attb[pl.ds(q * 16, 16)] for q in range(4)]

    for b in range(2):
        _zero_vmem(wbuf[b], 128)
    _zero_vmem(zb, 128)
    z16i = jnp.zeros((16,), I32)
    for b in range(2):
        for g in range(CB1 // 16 + 1):
            drs[b][pl.ds(min(g * 16, CB1 - 16), 16)] = z16i
    _zero_shared(zb, acc_sh, sid)
    plsc.subcore_barrier()

    nch = _EPW // CB1
    ebase = wid * _EPW

    def fire_idx(c, b):
        bs = _m8(ebase + c * CB1)
        pltpu.async_copy(src_hbm.at[pl.ds(bs, CB1)], si[b], semi[b])
        pltpu.async_copy(dst_hbm.at[pl.ds(bs, CB1)], di[b], semi[b])

    def wait_idx(b):
        pltpu.make_async_copy(src_hbm.at[pl.ds(0, CB1)], si[b], semi[b]).wait()
        pltpu.make_async_copy(dst_hbm.at[pl.ds(0, CB1)], di[b], semi[b]).wait()

    def fire_gather(c, b):
        pltpu.async_copy(tbl_hbm.at[si[b]], srow[b], semg[b])
        pltpu.async_copy(tbl_hbm.at[di[b]], drow[b], semg[b])

    def wait_gather(b):
        pltpu.make_async_copy(tbl_hbm.at[si[b]], srow[b], semg[b]).wait()
        pltpu.make_async_copy(tbl_hbm.at[di[b]], drow[b], semg[b]).wait()

    def wait_scatter(b):
        pltpu.make_async_copy(acc_hbm.at[pl.ds(0, CB1)], wbuf[b],
                              sems[b]).wait()

    for b in range(2):
        pltpu.async_copy(wbuf[b], acc_sh.at[drs[b]], sems[b], add=True)
        fire_idx(b, b)
    for b in range(2):
        wait_idx(b)
        fire_gather(b, b)

    def body(k2, _):
        bp = _m8((ebase + 2 * k2 * CB1) // 2)
        cpe = pltpu.async_copy(ep_hbm.at[pl.ds(bp, CB1)], epb, semp)
        for b in range(2):
            c = 2 * k2 + b
            wait_scatter(b)
            wait_gather(b)
            if b == 0:
                cpe.wait()
            for g in range(CB1 // 16 + 1):
                sl = pl.ds(min(g * 16, CB1 - 16), 16)
                drs[b][sl] = di[b][sl]
            cn = c + 2
            cc = jnp.where(cn < nch, cn, 0)
            fire_idx(cc, b)
            sr = srow[b]
            drr = drow[b]
            wb = wbuf[b]
            prow = b * (CB1 // 2)

            def quad(pq, _):
                for v in range(2):
                    p = 2 * pq + v
                    for u in range(2):
                        j = 2 * p + u
                        acc = _zeros16()
                        for q in range(4):
                            s = (sr[j, pl.ds(q * 16, 16)]
                                 + drr[j, pl.ds(64 + q * 16, 16)]
                                 + epb[prow + p, pl.ds(u * 64 + q * 16, 16)])
                            m = jnp.maximum(s, 0.2 * s)
                            acc = acc + m * att_vecs[q]
                        ex = jnp.exp(_allsum(acc))
                        for q in range(4):
                            sl = pl.ds(q * 16, 16)
                            wb[j, sl] = sr[j, sl] * ex
                        wb[j, pl.ds(64, 16)] = ex * mask0
                return 0
            lax.fori_loop(0, CB1 // 4, quad, 0)
            pltpu.async_copy(wb, acc_sh.at[drs[b]], sems[b], add=True)
            wait_idx(b)
            fire_gather(cc, b)
        return 0
    lax.fori_loop(0, nch // 2, body, 0)
    for b in range(2):
        wait_scatter(b)
        wait_gather(b)
    plsc.subcore_barrier()

    _copy_out_shared(acc_sh, acc_hbm, cid, sid)


def _s2(src, dst, tbl, ep, attf):
    return pl.kernel(
        _s2_body,
        out_type=jax.ShapeDtypeStruct((NC * N, 128), F32),
        mesh=plsc.VectorSubcoreMesh(**_MESH),
        scratch_types=[
            pltpu.VMEM((CB1,), I32),
            pltpu.VMEM((CB1,), I32),
            pltpu.VMEM((CB1,), I32),
            pltpu.VMEM((CB1,), I32),
            pltpu.VMEM((CB1,), I32),
            pltpu.VMEM((CB1,), I32),
            pltpu.VMEM((CB1, 128), F32),
            pltpu.VMEM((CB1, 128), F32),
            pltpu.VMEM((CB1, 128), F32),
            pltpu.VMEM((CB1, 128), F32),
            pltpu.VMEM((CB1, 128), F32),
            pltpu.VMEM((CB1, 128), F32),
            pltpu.VMEM((CB1, 128), F32),
            pltpu.VMEM((64,), F32),
            pltpu.VMEM((_SEGC, 128), F32),
            pltpu.VMEM_SHARED((N, 128), F32),
            pltpu.SemaphoreType.DMA,
            pltpu.SemaphoreType.DMA,
            pltpu.SemaphoreType.DMA,
            pltpu.SemaphoreType.DMA,
            pltpu.SemaphoreType.DMA,
            pltpu.SemaphoreType.DMA,
            pltpu.SemaphoreType.DMA,
        ],
    )(src, dst, tbl, ep, attf)


def _s3_body(src_hbm, dst_hbm, g_hbm, eec_hbm, cw_hbm,
             lg_hbm,
             si0, si1, di0, di1, srow0, srow1, drow0, drow1, ecb0, ecb1,
             wb_, lbuf, sem0, semi0, semi1, semg0, semg1):
    cid = lax.axis_index("c")
    sid = lax.axis_index("s")
    wid = cid * NS + sid
    lanes = lax.iota(I32, 16)

    si = (si0, si1)
    di = (di0, di1)
    srow = (srow0, srow1)
    drow = (drow0, drow1)
    ecb = (ecb0, ecb1)
    semi = (semi0, semi1)
    semg = (semg0, semg1)

    pltpu.sync_copy(cw_hbm, wb_)
    w_vecs = [wb_[pl.ds(q * 16, 16)] for q in range(4)]
    cb_vec = wb_[pl.ds(64, 16)]

    nch = _EPW // CB
    ebase = wid * _EPW

    def fire_idx(c, b):
        bs = _m8(ebase + c * CB)
        pltpu.async_copy(src_hbm.at[pl.ds(bs, CB)], si[b], semi[b])
        pltpu.async_copy(dst_hbm.at[pl.ds(bs, CB)], di[b], semi[b])
        pltpu.async_copy(eec_hbm.at[pl.ds(_m8(bs // 2), CB // 2)],
                         ecb[b], semi[b])

    def wait_idx(b):
        pltpu.make_async_copy(src_hbm.at[pl.ds(0, CB)], si[b], semi[b]).wait()
        pltpu.make_async_copy(dst_hbm.at[pl.ds(0, CB)], di[b], semi[b]).wait()
        pltpu.make_async_copy(eec_hbm.at[pl.ds(0, CB // 2)], ecb[b],
                              semi[b]).wait()

    def fire_gather(b):
        pltpu.async_copy(g_hbm.at[si[b]], srow[b], semg[b])
        pltpu.async_copy(g_hbm.at[di[b]], drow[b], semg[b])

    def wait_gather(b):
        pltpu.make_async_copy(g_hbm.at[pl.ds(0, CB)], srow[b], semg[b]).wait()
        pltpu.make_async_copy(g_hbm.at[pl.ds(0, CB)], drow[b], semg[b]).wait()

    def compute_write(c, b):
        sr = srow[b]
        drr = drow[b]
        ec = ecb[b]

        def group(g, _):
            lvec = cb_vec
            for jj in range(16):
                j = g * 16 + jj
                acc = _zeros16()
                for q in range(4):
                    z = (sr[j, pl.ds(q * 16, 16)]
                         + drr[j, pl.ds(64 + q * 16, 16)]
                         + ec[(g * 16 + jj) // 2,
                              pl.ds((jj % 2) * 64 + q * 16, 16)])
                    z = jnp.maximum(z, 0.0)
                    acc = acc + z * w_vecs[q]
                a = _allsum(acc)
                mj = jnp.where(lanes == jj, 1.0, 0.0).astype(F32)
                lvec = lvec + a * mj
            lbuf[pl.ds(g * 16, 16)] = lvec
            return 0
        lax.fori_loop(0, CB // 16, group, 0)
        pltpu.sync_copy(lbuf, lg_hbm.at[pl.ds(_m8(ebase + c * CB), CB)])

    fire_idx(0, 0)
    fire_idx(1, 1)
    wait_idx(0)
    fire_gather(0)
    wait_idx(1)
    fire_gather(1)
    wait_gather(0)
    compute_write(0, 0)
    fire_idx(2, 0)

    def body(k2, _):
        ca = 1 + 2 * k2
        wait_gather(1)
        wait_idx(0)
        fire_gather(0)
        compute_write(ca, 1)
        cn = ca + 2
        fire_idx(jnp.where(cn < nch, cn, 0), 1)
        wait_gather(0)
        wait_idx(1)
        fire_gather(1)
        compute_write(ca + 1, 0)
        cn2 = ca + 3
        fire_idx(jnp.where(cn2 < nch, cn2, 0), 0)
        return 0
    lax.fori_loop(0, (nch - 1) // 2, body, 0)
    wait_gather(1)
    wait_idx(0)


def _s3(src, dst, g, eec, cwm):
    return pl.kernel(
        _s3_body,
        out_type=jax.ShapeDtypeStruct((E,), F32),
        mesh=plsc.VectorSubcoreMesh(**_MESH),
        scratch_types=[
            pltpu.VMEM((CB,), I32),
            pltpu.VMEM((CB,), I32),
            pltpu.VMEM((CB,), I32),
            pltpu.VMEM((CB,), I32),
            pltpu.VMEM((CB, 128), F32),
            pltpu.VMEM((CB, 128), F32),
            pltpu.VMEM((CB, 128), F32),
            pltpu.VMEM((CB, 128), F32),
            pltpu.VMEM((CB // 2, 128), F32),
            pltpu.VMEM((CB // 2, 128), F32),
            pltpu.VMEM((80,), F32),
            pltpu.VMEM((CB,), F32),
            pltpu.SemaphoreType.DMA,
            pltpu.SemaphoreType.DMA,
            pltpu.SemaphoreType.DMA,
            pltpu.SemaphoreType.DMA,
            pltpu.SemaphoreType.DMA,
        ],
    )(src, dst, g, eec, cwm)


def _ep1_body(acca_ref, accb_ref, s0_ref, xlr_ref,
              ewT_ref, s16_ref, s64_ref, attm_ref, bias_ref,
              l2lwT_ref, l2lb_ref, l2rwT_ref, l2rb_ref,
              c2_ref, la_ref):
    s0s = s0_ref[0] + s0_ref[1]
    ea = s0s[:, :ED]
    cntv = jnp.dot(s0s, s16_ref[...], preferred_element_type=F32)
    la = ea / jnp.maximum(cntv, 1.0)
    la_ref[...] = la
    lp = jnp.dot(la, ewT_ref[...], preferred_element_type=F32)

    accs = [acca_ref[0], acca_ref[1], accb_ref[0], accb_ref[1]]
    xls = []
    mls = []
    for h in range(4):
        xlh = xlr_ref[h][:, :HID]
        xrh = xlr_ref[h][:, HID:]
        mm = xlh + xrh + lp[:, h * HID:(h + 1) * HID]
        xls.append(xlh)
        mls.append(jnp.maximum(mm, 0.2 * mm))
    al = sum(jnp.dot(mls[h], attm_ref[h * HID:(h + 1) * HID, :],
                     preferred_element_type=F32) for h in range(4))
    exl = jnp.exp(al)
    xl2v = l2lb_ref[...]
    xr2v = l2rb_ref[...]
    for h in range(4):
        exh = exl[:, h:h + 1]
        denh = jnp.dot(accs[h], s64_ref[...], preferred_element_type=F32)
        num = accs[h][:, :HID] + exh * xls[h]
        hv = num / (denh + exh + 1e-16) + bias_ref[:, h * HID:(h + 1) * HID]
        hv = jnp.where(hv > 0, hv, jnp.exp(jnp.minimum(hv, 0.0)) - 1.0)
        xl2v = xl2v + jnp.dot(hv, l2lwT_ref[h * HID:(h + 1) * HID, :],
                              preferred_element_type=F32)
        xr2v = xr2v + jnp.dot(hv, l2rwT_ref[h * HID:(h + 1) * HID, :],
                              preferred_element_type=F32)
    c2_ref[...] = jnp.concatenate([xl2v, xr2v], axis=1)


def _ep1(acca, accb, s0acc, xlr, ewT, s16, s64, attm, bias,
         l2lwT, l2lb, l2rwT, l2rb):
    def full(shape):
        return pl.BlockSpec(shape, lambda i, _n=len(shape): (0,) * _n)
    return pl.pallas_call(
        _ep1_body,
        grid=(N // BN,),
        in_specs=[
            pl.BlockSpec((NC, BN, 128), lambda i: (0, i, 0)),
            pl.BlockSpec((NC, BN, 128), lambda i: (0, i, 0)),
            pl.BlockSpec((NC, BN, 128), lambda i: (0, i, 0)),
            pl.BlockSpec((4, BN, 128), lambda i: (0, i, 0)),
            full((ED, 256)),
            full((128, 1)),
            full((128, 1)),
            full((256, 4)),
            full((1, 256)),
            full((256, HID)),
            full((1, HID)),
            full((256, HID)),
            full((1, HID)),
        ],
        out_specs=[pl.BlockSpec((BN, 128), lambda i: (i, 0)),
                   pl.BlockSpec((BN, ED), lambda i: (i, 0))],
        out_shape=[jax.ShapeDtypeStruct((N, 128), F32),
                   jax.ShapeDtypeStruct((N, ED), F32)],
    )(acca, accb, s0acc, xlr, ewT, s16, s64, attm, bias,
      l2lwT, l2lb, l2rwT, l2rb)


def _ep2_body(acc_ref, c2_ref, la_ref,
              ew2T_ref, att2_ref, s64_ref, bias_ref, wsT_ref, wdT_ref,
              g_ref):
    accs = acc_ref[0] + acc_ref[1]
    den = jnp.dot(accs, s64_ref[...], preferred_element_type=F32)
    lp2 = jnp.dot(la_ref[...], ew2T_ref[...], preferred_element_type=F32)
    xl2 = c2_ref[:, :HID]
    xr2 = c2_ref[:, HID:]
    mm = xl2 + xr2 + lp2
    ml = jnp.maximum(mm, 0.2 * mm)
    al = jnp.dot(ml, att2_ref[...], preferred_element_type=F32)
    ex = jnp.exp(al)
    h2 = (accs[:, :HID] + ex * xl2) / (den + ex + 1e-16) + bias_ref[...]
    h2 = jnp.where(h2 > 0, h2, jnp.exp(jnp.minimum(h2, 0.0)) - 1.0)
    g_ref[...] = jnp.concatenate(
        [jnp.dot(h2, wsT_ref[...], preferred_element_type=F32),
         jnp.dot(h2, wdT_ref[...], preferred_element_type=F32)], axis=1)


def _ep2(acc, c2, la, ew2T, att2, s64, bias, wsT, wdT):
    def full(shape):
        return pl.BlockSpec(shape, lambda i, _n=len(shape): (0,) * _n)
    return pl.pallas_call(
        _ep2_body,
        grid=(N // BN,),
        in_specs=[
            pl.BlockSpec((NC, BN, 128), lambda i: (0, i, 0)),
            pl.BlockSpec((BN, 128), lambda i: (i, 0)),
            pl.BlockSpec((BN, ED), lambda i: (i, 0)),
            full((ED, HID)),
            full((HID, 1)),
            full((128, 1)),
            full((1, HID)),
            full((HID, HID)),
            full((HID, HID)),
        ],
        out_specs=pl.BlockSpec((BN, 128), lambda i: (i, 0)),
        out_shape=jax.ShapeDtypeStruct((N, 128), F32),
    )(acc, c2, la, ew2T, att2, s64, bias, wsT, wdT)


_S16 = np.zeros((128, 1), np.float32)
_S16[16, 0] = 1.0
_S64 = np.zeros((128, 1), np.float32)
_S64[64, 0] = 1.0


def kernel(x, edge_index, edge_attr, l1_lw, l1_lb, l1_rw, l1_rb, l1_ew,
           l1_att, l1_bias, l2_lw, l2_lb, l2_rw, l2_rb, l2_ew, l2_att,
           l2_bias, c1_w, c1_b, c2_w, c2_b):
    src = edge_index[0]
    dst = edge_index[1]
    eae = edge_attr[0::2]
    eao = edge_attr[1::2]
    eaf = edge_attr.reshape(-1)

    def hsplit(w):
        return w.reshape(w.shape[0], 4, HID).transpose(1, 0, 2)

    tbl1 = _t1(x, hsplit(l1_lw.T), l1_lb.reshape(4, 1, HID),
               hsplit(l1_rw.T), l1_rb.reshape(4, 1, HID))
    ep1 = _t2a(eae, eao, hsplit(l1_ew.T))
    ep2, eec = _t2b(eae, eao, l2_ew.T, c1_w[:, 128:].T, c1_b[None])

    s0acc = _s0(dst, eaf)

    tblf = tbl1.reshape(4 * N, 128)
    epf = ep1.reshape(4 * (E // 2), 128)
    attm1 = l1_att.reshape(-1)
    hoff = (jnp.arange(4, dtype=I32) * N)[:, None]
    srcoff = (src[None, :] + hoff).reshape(-1)
    dstoff = (dst[None, :] + hoff).reshape(-1)
    acca = _s1h(0, srcoff, dstoff, tblf, epf, attm1)
    accb = _s1h(1, srcoff, dstoff, tblf, epf, attm1)

    attmask = jnp.zeros((256, 4), F32).at[
        jnp.arange(256), jnp.arange(256) // 64].set(l1_att.reshape(-1))
    c2tbl, la = _ep1(
        acca.reshape(NC, N, 128), accb.reshape(NC, N, 128),
        s0acc.reshape(NC, N, 128), tbl1,
        l1_ew.T, jnp.asarray(_S16), jnp.asarray(_S64),
        attmask, l1_bias[None],
        l2_lw.T, l2_lb[None], l2_rw.T, l2_rb[None])

    acc2 = _s2(src, dst, c2tbl, ep2, l2_att.reshape(-1))

    gtbl = _ep2(acc2.reshape(NC, N, 128), c2tbl, la,
                l2_ew.T, l2_att.reshape(HID, 1), jnp.asarray(_S64),
                l2_bias[None], c1_w[:, :64].T, c1_w[:, 64:128].T)

    cwm = jnp.concatenate([c2_w.reshape(-1),
                           jnp.broadcast_to(c2_b, (16,))])
    logits = _s3(src, dst, gtbl, eec, cwm)
    return logits
